# Initial kernel scaffold; baseline (speedup 1.0000x reference)
import jax
import jax.numpy as jnp
from jax.experimental import pallas as pl


def kernel(edge_index, edge_type, embeddings, W0, b0):
    raise NotImplementedError("write your pallas kernel here")


# speedup vs baseline: 7.1791x; 7.1791x over previous
import functools

import jax
import jax.numpy as jnp
from jax import lax
from jax.experimental import pallas as pl
from jax.experimental.pallas import tpu as pltpu
from jax.experimental.pallas import tpu_sc as plsc

NN = 10000
NR = 5
KK = 2 * NR
ED = 128
NC, NS, LL = 2, 16, 16
NW = NC * NS
CHUNK = 128
RPT = 160
NOP = NW * RPT * CHUNK
NOPROWS = NOP // CHUNK
CPAD = KK * NN + 96
ZR = CPAD // NS
HSTRIPE = 624
_SEGS = ((0, 128), (128, 128), (256, 128), (384, 128), (512, 112))
ORDER = (0, 5, 1, 6, 2, 7, 3, 8, 4, 9)
_IBLK = 32

_mesh = plsc.VectorSubcoreMesh(core_axis_name="c", subcore_axis_name="s")


_BN1 = 400
_NB1 = NN // _BN1


def _mm_body(emb_ref, w_ref, b_ref, y_ref):
    y_ref[...] = (
        jnp.dot(emb_ref[...], w_ref[0], preferred_element_type=jnp.float32)
        + b_ref[0]
    )


def _mm_call(emb, W0, b0):
    return pl.pallas_call(
        _mm_body,
        grid=(KK, _NB1),
        in_specs=[
            pl.BlockSpec((_BN1, ED), lambda k, i: (i, 0)),
            pl.BlockSpec((1, ED, ED), lambda k, i: (k, 0, 0)),
            pl.BlockSpec((1, 1, ED), lambda k, i: (k, 0, 0)),
        ],
        out_specs=pl.BlockSpec((_BN1, ED), lambda k, i: (k * _NB1 + i, 0)),
        out_shape=jax.ShapeDtypeStruct((KK * NN, ED), jnp.float32),
    )(emb, W0, b0.reshape(KK, 1, ED))


@functools.partial(
    pl.kernel,
    out_type=jax.ShapeDtypeStruct((NC * CPAD,), jnp.float32),
    mesh=_mesh,
    scratch_types=[
        pltpu.VMEM((RPT, CHUNK), jnp.int32),
        pltpu.VMEM((CHUNK,), jnp.float32),
        pltpu.VMEM((ZR,), jnp.float32),
        pltpu.VMEM_SHARED((CPAD,), jnp.float32)
    ],
)
def _count_kernel(widx_hbm, out_hbm, idxbuf, ones_v, zbuf, c_sh):
    cid = lax.axis_index("c")
    sid = lax.axis_index("s")
    wid = sid * NC + cid
    zero16 = jnp.zeros((16,), jnp.float32)
    one16 = jnp.ones((16,), jnp.float32)

    def _zb(i, carry):
        zbuf[pl.ds(i * 16, 16)] = zero16
        return carry

    lax.fori_loop(0, ZR // 16, _zb, 0)
    for i in range(CHUNK // 16):
        ones_v[pl.ds(i * 16, 16)] = one16
    pltpu.sync_copy(zbuf, c_sh.at[pl.ds(sid * ZR, ZR)])
    plsc.subcore_barrier()

    pltpu.sync_copy(widx_hbm.at[pl.ds(wid * RPT, RPT)], idxbuf)

    def _body(j, carry):
        pltpu.sync_copy(ones_v, c_sh.at[idxbuf.at[j]], add=True)
        return carry

    lax.fori_loop(0, RPT, _body, 0)
    plsc.subcore_barrier()
    pltpu.sync_copy(c_sh.at[pl.ds(sid * ZR, ZR)], zbuf)
    pltpu.sync_copy(zbuf, out_hbm.at[pl.ds(cid * CPAD + sid * ZR, ZR)])


_BN3 = 2000
_NB3 = NN // _BN3


def _wt_body(cp_ref, w_ref, wemb_ref):
    c = cp_ref[0] + cp_ref[1]
    a = 2.0 / jnp.maximum(c, 1.0)
    p = jnp.ones((_BN3, 1), jnp.float32)
    cols = [None] * KK
    for j in reversed(range(KK)):
        kj = ORDER[j]
        p = p * a[:, kj:kj + 1]
        cols[kj] = p
    w_ref[...] = jnp.concatenate(cols, axis=1)
    wemb_ref[...] = p


def _wt_call(cpr):
    return pl.pallas_call(
        _wt_body,
        grid=(_NB3,),
        in_specs=[pl.BlockSpec((NC, _BN3, KK), lambda i: (0, i, 0))],
        out_specs=[
            pl.BlockSpec((_BN3, KK), lambda i: (i, 0)),
            pl.BlockSpec((_BN3, 1), lambda i: (i, 0)),
        ],
        out_shape=[
            jax.ShapeDtypeStruct((NN, KK), jnp.float32),
            jax.ShapeDtypeStruct((NN, 1), jnp.float32),
        ],
    )(cpr)


_GDN = lax.GatherDimensionNumbers(
    offset_dims=(), collapsed_slice_dims=(0,), start_index_map=(0,))


def _bcast_lane(v16, i):
    return lax.gather(
        v16, jnp.full((16, 1), i, jnp.int32), _GDN, slice_sizes=(1,),
        mode=lax.GatherScatterMode.PROMISE_IN_BOUNDS)


@functools.partial(
    pl.kernel,
    out_type=jax.ShapeDtypeStruct((NC, NN, ED), jnp.float32),
    mesh=_mesh,
    scratch_types=[
        pltpu.VMEM((_IBLK, CHUNK), jnp.int32),
        pltpu.VMEM((_IBLK, CHUNK), jnp.int32),
        pltpu.VMEM((_IBLK, CHUNK), jnp.int32),
        pltpu.VMEM((CHUNK, ED), jnp.float32),
        pltpu.VMEM((CHUNK,), jnp.float32),
        pltpu.VMEM_SHARED((NN, ED), jnp.float32),
    ],
)
def _scatter_kernel(yf_hbm, wflat_hbm, gidx_hbm, widx_hbm, didx_hbm, out_hbm,
                    gbuf, wibuf, dbuf, rows, wvals, h_sh):
    cid = lax.axis_index("c")
    sid = lax.axis_index("s")
    wid = sid * NC + cid
    zero16 = jnp.zeros((16,), jnp.float32)

    def _zrow(r, carry):
        for cb in range(ED // 16):
            rows[r, pl.ds(cb * 16, 16)] = zero16
        return carry

    lax.fori_loop(0, CHUNK, _zrow, 0)
    hbase = sid * HSTRIPE
    for off, sz in _SEGS:
        pltpu.sync_copy(rows.at[pl.ds(0, sz)],
                        h_sh.at[pl.ds(hbase + off, sz)])

    @pl.when(sid == NS - 1)
    def _zero_tail():
        pltpu.sync_copy(rows.at[pl.ds(0, 16)], h_sh.at[pl.ds(NN - 16, 16)])

    plsc.subcore_barrier()

    def _iblk(bi, carry):
        rb = wid * RPT + bi * _IBLK
        pltpu.sync_copy(gidx_hbm.at[pl.ds(rb, _IBLK)], gbuf)
        pltpu.sync_copy(widx_hbm.at[pl.ds(rb, _IBLK)], wibuf)
        pltpu.sync_copy(didx_hbm.at[pl.ds(rb, _IBLK)], dbuf)

        def _chunk(j, c1):
            pltpu.sync_copy(yf_hbm.at[gbuf.at[j]], rows)
            pltpu.sync_copy(wflat_hbm.at[wibuf.at[j]], wvals)

            def _grp(g, c2):
                wv = wvals[pl.ds(g * 16, 16)]
                for i in range(16):
                    wb = _bcast_lane(wv, i)
                    e = g * 16 + i
                    for cb in range(ED // 16):
                        sl = pl.ds(cb * 16, 16)
                        rows[e, sl] = rows[e, sl] * wb
                return c2

            lax.fori_loop(0, CHUNK // 16, _grp, 0)
            pltpu.sync_copy(rows, h_sh.at[dbuf.at[j]], add=True)
            return c1

        lax.fori_loop(0, _IBLK, _chunk, 0)
        return carry

    lax.fori_loop(0, RPT // _IBLK, _iblk, 0)
    plsc.subcore_barrier()
    for off, sz in _SEGS:
        sl = pl.ds(hbase + off, sz)
        pltpu.sync_copy(h_sh.at[sl], rows.at[pl.ds(0, sz)])
        pltpu.sync_copy(rows.at[pl.ds(0, sz)], out_hbm.at[cid, sl])

    @pl.when(sid == NS - 1)
    def _out_tail():
        sl = pl.ds(NN - 16, 16)
        pltpu.sync_copy(h_sh.at[sl], rows.at[pl.ds(0, 16)])
        pltpu.sync_copy(rows.at[pl.ds(0, 16)], out_hbm.at[cid, sl])


_BN5 = 400
_NB5 = NN // _BN5


def _comb_body(emb_ref, wemb_ref, hp_ref, out_ref):
    out_ref[...] = emb_ref[...] * wemb_ref[...] + hp_ref[0] + hp_ref[1]


def _comb_call(emb, wemb, hp):
    return pl.pallas_call(
        _comb_body,
        grid=(_NB5,),
        in_specs=[
            pl.BlockSpec((_BN5, ED), lambda i: (i, 0)),
            pl.BlockSpec((_BN5, 1), lambda i: (i, 0)),
            pl.BlockSpec((NC, _BN5, ED), lambda i: (0, i, 0)),
        ],
        out_specs=pl.BlockSpec((_BN5, ED), lambda i: (i, 0)),
        out_shape=jax.ShapeDtypeStruct((NN, ED), jnp.float32),
    )(emb, wemb, hp)


def kernel(edge_index, edge_type, embeddings, W0, b0):
    ne = edge_index.shape[1]
    t = edge_type.astype(jnp.int32)
    ei0 = edge_index[0].astype(jnp.int32)
    ei1 = edge_index[1].astype(jnp.int32)

    gidx = jnp.concatenate([t * NN + ei1, (t + NR) * NN + ei0])
    widx = jnp.concatenate([ei0 * KK + t, ei1 * KK + (t + NR)])
    didx = jnp.concatenate([ei0, ei1])
    pad = NOP - 2 * ne
    gidx = jnp.concatenate([gidx, jnp.zeros((pad,), jnp.int32)])
    widx = jnp.concatenate([widx, jnp.full((pad,), KK * NN, jnp.int32)])
    didx = jnp.concatenate([didx, jnp.zeros((pad,), jnp.int32)])
    gidx2 = gidx.reshape(NOPROWS, CHUNK)
    widx2 = widx.reshape(NOPROWS, CHUNK)
    didx2 = didx.reshape(NOPROWS, CHUNK)

    yf = _mm_call(embeddings, W0, b0)
    cp = _count_kernel(widx2).reshape(NC, CPAD)
    cpr = cp[:, :KK * NN].reshape(NC, NN, KK)
    wT, wemb = _wt_call(cpr)
    wflat = jnp.concatenate(
        [wT.reshape(-1), jnp.zeros((CPAD - KK * NN,), jnp.float32)])
    hp = _scatter_kernel(yf, wflat, gidx2, widx2, didx2)
    return _comb_call(embeddings, wemb, hp)


# speedup vs baseline: 9.6392x; 1.3427x over previous
import functools

import jax
import jax.numpy as jnp
from jax import lax
from jax.experimental import pallas as pl
from jax.experimental.pallas import tpu as pltpu
from jax.experimental.pallas import tpu_sc as plsc

NN = 10000
NR = 5
KK = 2 * NR
ED = 128
NC, NS, LL = 2, 16, 16
NW = NC * NS
CHUNK = 128
RPT = 160
NOP = NW * RPT * CHUNK
NOPROWS = NOP // CHUNK
CPAD = KK * NN + 96
ZR = CPAD // NS
HSTRIPE = 624
_SEGS = ((0, 128), (128, 128), (256, 128), (384, 128), (512, 112))
ORDER = (0, 5, 1, 6, 2, 7, 3, 8, 4, 9)
_IBLK = 32

_mesh = plsc.VectorSubcoreMesh(core_axis_name="c", subcore_axis_name="s")


_BN1 = 400
_NB1 = NN // _BN1


def _mm_body(emb_ref, w_ref, b_ref, y_ref):
    y_ref[...] = (
        jnp.dot(emb_ref[...], w_ref[0], preferred_element_type=jnp.float32)
        + b_ref[0]
    )


def _mm_call(emb, W0, b0):
    return pl.pallas_call(
        _mm_body,
        grid=(KK, _NB1),
        in_specs=[
            pl.BlockSpec((_BN1, ED), lambda k, i: (i, 0)),
            pl.BlockSpec((1, ED, ED), lambda k, i: (k, 0, 0)),
            pl.BlockSpec((1, 1, ED), lambda k, i: (k, 0, 0)),
        ],
        out_specs=pl.BlockSpec((_BN1, ED), lambda k, i: (k * _NB1 + i, 0)),
        out_shape=jax.ShapeDtypeStruct((KK * NN, ED), jnp.float32),
    )(emb, W0, b0.reshape(KK, 1, ED))


@functools.partial(
    pl.kernel,
    out_type=jax.ShapeDtypeStruct((NC * CPAD,), jnp.float32),
    mesh=_mesh,
    scratch_types=[
        pltpu.VMEM((RPT, CHUNK), jnp.int32),
        pltpu.VMEM((CHUNK,), jnp.float32),
        pltpu.VMEM((ZR,), jnp.float32),
        pltpu.VMEM_SHARED((CPAD,), jnp.float32)
    ],
)
def _count_kernel(widx_hbm, out_hbm, idxbuf, ones_v, zbuf, c_sh):
    cid = lax.axis_index("c")
    sid = lax.axis_index("s")
    wid = sid * NC + cid
    zero16 = jnp.zeros((16,), jnp.float32)
    one16 = jnp.ones((16,), jnp.float32)

    def _zb(i, carry):
        zbuf[pl.ds(i * 16, 16)] = zero16
        return carry

    lax.fori_loop(0, ZR // 16, _zb, 0)
    for i in range(CHUNK // 16):
        ones_v[pl.ds(i * 16, 16)] = one16
    pltpu.sync_copy(zbuf, c_sh.at[pl.ds(sid * ZR, ZR)])
    plsc.subcore_barrier()

    pltpu.sync_copy(widx_hbm.at[pl.ds(wid * RPT, RPT)], idxbuf)

    def _body(j, carry):
        pltpu.sync_copy(ones_v, c_sh.at[idxbuf.at[j]], add=True)
        return carry

    lax.fori_loop(0, RPT, _body, 0)
    plsc.subcore_barrier()
    pltpu.sync_copy(c_sh.at[pl.ds(sid * ZR, ZR)], zbuf)
    pltpu.sync_copy(zbuf, out_hbm.at[pl.ds(cid * CPAD + sid * ZR, ZR)])


_BN3 = 2000
_NB3 = NN // _BN3


def _wt_body(cp_ref, w_ref, wemb_ref):
    c = cp_ref[0] + cp_ref[1]
    a = 2.0 / jnp.maximum(c, 1.0)
    p = jnp.ones((_BN3, 1), jnp.float32)
    cols = [None] * KK
    for j in reversed(range(KK)):
        kj = ORDER[j]
        p = p * a[:, kj:kj + 1]
        cols[kj] = p
    w_ref[...] = jnp.concatenate(cols, axis=1)
    wemb_ref[...] = p


def _wt_call(cpr):
    return pl.pallas_call(
        _wt_body,
        grid=(_NB3,),
        in_specs=[pl.BlockSpec((NC, _BN3, KK), lambda i: (0, i, 0))],
        out_specs=[
            pl.BlockSpec((_BN3, KK), lambda i: (i, 0)),
            pl.BlockSpec((_BN3, 1), lambda i: (i, 0)),
        ],
        out_shape=[
            jax.ShapeDtypeStruct((NN, KK), jnp.float32),
            jax.ShapeDtypeStruct((NN, 1), jnp.float32),
        ],
    )(cpr)


_GDN = lax.GatherDimensionNumbers(
    offset_dims=(), collapsed_slice_dims=(0,), start_index_map=(0,))


def _bcast_lane(v16, i):
    return lax.gather(
        v16, jnp.full((16, 1), i, jnp.int32), _GDN, slice_sizes=(1,),
        mode=lax.GatherScatterMode.PROMISE_IN_BOUNDS)


@functools.partial(
    pl.kernel,
    out_type=jax.ShapeDtypeStruct((NC, NN, ED), jnp.float32),
    mesh=_mesh,
    scratch_types=[
        pltpu.VMEM((_IBLK, CHUNK), jnp.int32),
        pltpu.VMEM((_IBLK, CHUNK), jnp.int32),
        pltpu.VMEM((_IBLK, CHUNK), jnp.int32),
        pltpu.VMEM((2, CHUNK, ED), jnp.float32),
        pltpu.VMEM((2, CHUNK), jnp.float32),
        pltpu.VMEM_SHARED((NN, ED), jnp.float32),
        pltpu.SemaphoreType.DMA,
        pltpu.SemaphoreType.DMA,
        pltpu.SemaphoreType.DMA,
        pltpu.SemaphoreType.DMA,
        pltpu.SemaphoreType.DMA,
        pltpu.SemaphoreType.DMA,
    ],
)
def _scatter_kernel(yf_hbm, wflat_hbm, gidx_hbm, widx_hbm, didx_hbm, out_hbm,
                    gbuf, wibuf, dbuf, rows, wvals, h_sh,
                    sg0, sg1, sw0, sw1, ss0, ss1):
    cid = lax.axis_index("c")
    sid = lax.axis_index("s")
    wid = sid * NC + cid
    zero16 = jnp.zeros((16,), jnp.float32)
    sg = (sg0, sg1)
    sw = (sw0, sw1)
    ss = (ss0, ss1)

    def _issue_gather(jj, b):
        pltpu.async_copy(yf_hbm.at[gbuf.at[jj]], rows.at[b], sg[b])
        pltpu.async_copy(wflat_hbm.at[wibuf.at[jj]], wvals.at[b], sw[b])

    def _wait_gather(b):
        pltpu.make_async_copy(yf_hbm.at[gbuf.at[0]], rows.at[b], sg[b]).wait()
        pltpu.make_async_copy(wflat_hbm.at[wibuf.at[0]], wvals.at[b],
                              sw[b]).wait()

    def _issue_scatter(jj, b):
        pltpu.async_copy(rows.at[b], h_sh.at[dbuf.at[jj]], ss[b], add=True)

    def _wait_scatter(b):
        pltpu.make_async_copy(rows.at[b], h_sh.at[dbuf.at[0]], ss[b]).wait()

    def _scale(b):
        def _grp(g, c2):
            wv = wvals[b, pl.ds(g * 16, 16)]
            for i in range(16):
                wb = _bcast_lane(wv, i)
                e = g * 16 + i
                for cb in range(ED // 16):
                    sl = pl.ds(cb * 16, 16)
                    rows[b, e, sl] = rows[b, e, sl] * wb
            return c2

        lax.fori_loop(0, CHUNK // 16, _grp, 0)

    def _zrow(r, carry):
        for cb in range(ED // 16):
            rows[0, r, pl.ds(cb * 16, 16)] = zero16
        return carry

    lax.fori_loop(0, CHUNK, _zrow, 0)
    hbase = sid * HSTRIPE
    for off, sz in _SEGS:
        pltpu.sync_copy(rows.at[0, pl.ds(0, sz)],
                        h_sh.at[pl.ds(hbase + off, sz)])

    @pl.when(sid == NS - 1)
    def _zero_tail():
        pltpu.sync_copy(rows.at[0, pl.ds(0, 16)], h_sh.at[pl.ds(NN - 16, 16)])

    plsc.subcore_barrier()

    def _iblk(bi, carry):
        rb = wid * RPT + bi * _IBLK

        @pl.when(bi >= 1)
        def _wait_prev_tail():
            _wait_scatter(1)

        pltpu.sync_copy(gidx_hbm.at[pl.ds(rb, _IBLK)], gbuf)
        pltpu.sync_copy(widx_hbm.at[pl.ds(rb, _IBLK)], wibuf)
        pltpu.sync_copy(didx_hbm.at[pl.ds(rb, _IBLK)], dbuf)
        _issue_gather(0, 0)

        def _pair(p, c1):
            @pl.when(p >= 1)
            def _w0():
                _wait_scatter(1)

            _issue_gather(2 * p + 1, 1)
            _wait_gather(0)
            _scale(0)
            _issue_scatter(2 * p, 0)
            _wait_scatter(0)

            @pl.when(p <= _IBLK // 2 - 2)
            def _pf1():
                _issue_gather(2 * p + 2, 0)

            _wait_gather(1)
            _scale(1)
            _issue_scatter(2 * p + 1, 1)
            return c1

        lax.fori_loop(0, _IBLK // 2, _pair, 0)
        return carry

    lax.fori_loop(0, RPT // _IBLK, _iblk, 0)
    _wait_scatter(1)
    plsc.subcore_barrier()
    for off, sz in _SEGS:
        sl = pl.ds(hbase + off, sz)
        pltpu.sync_copy(h_sh.at[sl], rows.at[0, pl.ds(0, sz)])
        pltpu.sync_copy(rows.at[0, pl.ds(0, sz)], out_hbm.at[cid, sl])

    @pl.when(sid == NS - 1)
    def _out_tail():
        sl = pl.ds(NN - 16, 16)
        pltpu.sync_copy(h_sh.at[sl], rows.at[0, pl.ds(0, 16)])
        pltpu.sync_copy(rows.at[0, pl.ds(0, 16)], out_hbm.at[cid, sl])


_BN5 = 400
_NB5 = NN // _BN5


def _comb_body(emb_ref, wemb_ref, hp_ref, out_ref):
    out_ref[...] = emb_ref[...] * wemb_ref[...] + hp_ref[0] + hp_ref[1]


def _comb_call(emb, wemb, hp):
    return pl.pallas_call(
        _comb_body,
        grid=(_NB5,),
        in_specs=[
            pl.BlockSpec((_BN5, ED), lambda i: (i, 0)),
            pl.BlockSpec((_BN5, 1), lambda i: (i, 0)),
            pl.BlockSpec((NC, _BN5, ED), lambda i: (0, i, 0)),
        ],
        out_specs=pl.BlockSpec((_BN5, ED), lambda i: (i, 0)),
        out_shape=jax.ShapeDtypeStruct((NN, ED), jnp.float32),
    )(emb, wemb, hp)


def kernel(edge_index, edge_type, embeddings, W0, b0):
    ne = edge_index.shape[1]
    t = edge_type.astype(jnp.int32)
    ei0 = edge_index[0].astype(jnp.int32)
    ei1 = edge_index[1].astype(jnp.int32)

    gidx = jnp.concatenate([t * NN + ei1, (t + NR) * NN + ei0])
    widx = jnp.concatenate([ei0 * KK + t, ei1 * KK + (t + NR)])
    didx = jnp.concatenate([ei0, ei1])
    pad = NOP - 2 * ne
    gidx = jnp.concatenate([gidx, jnp.zeros((pad,), jnp.int32)])
    widx = jnp.concatenate([widx, jnp.full((pad,), KK * NN, jnp.int32)])
    didx = jnp.concatenate([didx, jnp.zeros((pad,), jnp.int32)])
    gidx2 = gidx.reshape(NOPROWS, CHUNK)
    widx2 = widx.reshape(NOPROWS, CHUNK)
    didx2 = didx.reshape(NOPROWS, CHUNK)

    yf = _mm_call(embeddings, W0, b0)
    cp = _count_kernel(widx2).reshape(NC, CPAD)
    cpr = cp[:, :KK * NN].reshape(NC, NN, KK)
    wT, wemb = _wt_call(cpr)
    wflat = jnp.concatenate(
        [wT.reshape(-1), jnp.zeros((CPAD - KK * NN,), jnp.float32)])
    hp = _scatter_kernel(yf, wflat, gidx2, widx2, didx2)
    return _comb_call(embeddings, wemb, hp)


# speedup vs baseline: 17.1547x; 1.7797x over previous
import functools

import jax
import jax.numpy as jnp
from jax import lax
from jax.experimental import pallas as pl
from jax.experimental.pallas import tpu as pltpu
from jax.experimental.pallas import tpu_sc as plsc

NN = 10000
NR = 5
KK = 2 * NR
ED = 128
NC, NS, LL = 2, 16, 16
NW = NC * NS
CHUNK = 128
RPT = 160
NOP = NW * RPT * CHUNK
NOPROWS = NOP // CHUNK
CPAD = KK * NN + 96
ZR = CPAD // NS
HSTRIPE = 624
_SEGS = ((0, 128), (128, 128), (256, 128), (384, 128), (512, 112))
ORDER = (0, 5, 1, 6, 2, 7, 3, 8, 4, 9)
_IBLK = 8
_R0, _R1 = 160, 160

_mesh = plsc.VectorSubcoreMesh(core_axis_name="c", subcore_axis_name="s")


_BN1 = 400
_NB1 = NN // _BN1


def _mm_body(emb_ref, w_ref, b_ref, y_ref):
    y_ref[...] = (
        jnp.dot(emb_ref[...], w_ref[0], preferred_element_type=jnp.float32)
        + b_ref[0]
    )


def _mm_call(emb, W0, b0):
    return pl.pallas_call(
        _mm_body,
        grid=(KK, _NB1),
        in_specs=[
            pl.BlockSpec((_BN1, ED), lambda k, i: (i, 0)),
            pl.BlockSpec((1, ED, ED), lambda k, i: (k, 0, 0)),
            pl.BlockSpec((1, 1, ED), lambda k, i: (k, 0, 0)),
        ],
        out_specs=pl.BlockSpec((_BN1, ED), lambda k, i: (k * _NB1 + i, 0)),
        out_shape=jax.ShapeDtypeStruct((KK * NN, ED), jnp.float32),
    )(emb, W0, b0.reshape(KK, 1, ED))


@functools.partial(
    pl.kernel,
    out_type=jax.ShapeDtypeStruct((NC * CPAD,), jnp.float32),
    mesh=_mesh,
    scratch_types=[
        pltpu.VMEM((RPT, CHUNK), jnp.int32),
        pltpu.VMEM((CHUNK,), jnp.float32),
        pltpu.VMEM((ZR,), jnp.float32),
        pltpu.VMEM_SHARED((CPAD,), jnp.float32)
    ],
)
def _count_kernel(widx_hbm, out_hbm, idxbuf, ones_v, zbuf, c_sh):
    cid = lax.axis_index("c")
    sid = lax.axis_index("s")
    wid = sid * NC + cid
    zero16 = jnp.zeros((16,), jnp.float32)
    one16 = jnp.ones((16,), jnp.float32)

    def _zb(i, carry):
        zbuf[pl.ds(i * 16, 16)] = zero16
        return carry

    lax.fori_loop(0, ZR // 16, _zb, 0)
    for i in range(CHUNK // 16):
        ones_v[pl.ds(i * 16, 16)] = one16
    pltpu.sync_copy(zbuf, c_sh.at[pl.ds(sid * ZR, ZR)])
    plsc.subcore_barrier()

    pltpu.sync_copy(widx_hbm.at[pl.ds(wid * RPT, RPT)], idxbuf)

    def _body(j, carry):
        pltpu.sync_copy(ones_v, c_sh.at[idxbuf.at[j]], add=True)
        return carry

    lax.fori_loop(0, RPT, _body, 0)
    plsc.subcore_barrier()
    pltpu.sync_copy(c_sh.at[pl.ds(sid * ZR, ZR)], zbuf)
    pltpu.sync_copy(zbuf, out_hbm.at[pl.ds(cid * CPAD + sid * ZR, ZR)])


_BN3 = 2000
_NB3 = NN // _BN3


def _wt_body(cp_ref, w_ref, wemb_ref):
    c = cp_ref[0] + cp_ref[1]
    a = 2.0 / jnp.maximum(c, 1.0)
    p = jnp.ones((_BN3, 1), jnp.float32)
    cols = [None] * KK
    for j in reversed(range(KK)):
        kj = ORDER[j]
        p = p * a[:, kj:kj + 1]
        cols[kj] = p
    w_ref[...] = jnp.concatenate(cols, axis=1)
    wemb_ref[...] = p


def _wt_call(cpr):
    return pl.pallas_call(
        _wt_body,
        grid=(_NB3,),
        in_specs=[pl.BlockSpec((NC, _BN3, KK), lambda i: (0, i, 0))],
        out_specs=[
            pl.BlockSpec((_BN3, KK), lambda i: (i, 0)),
            pl.BlockSpec((_BN3, 1), lambda i: (i, 0)),
        ],
        out_shape=[
            jax.ShapeDtypeStruct((NN, KK), jnp.float32),
            jax.ShapeDtypeStruct((NN, 1), jnp.float32),
        ],
    )(cpr)


_GDN = lax.GatherDimensionNumbers(
    offset_dims=(), collapsed_slice_dims=(0,), start_index_map=(0,))


def _bcast_lane(v16, i):
    return lax.gather(
        v16, jnp.full((16, 1), i, jnp.int32), _GDN, slice_sizes=(1,),
        mode=lax.GatherScatterMode.PROMISE_IN_BOUNDS)


@functools.partial(
    pl.kernel,
    out_type=jax.ShapeDtypeStruct((NC, NN, ED), jnp.float32),
    mesh=_mesh,
    scratch_types=[
        pltpu.VMEM((_IBLK, CHUNK), jnp.int32),
        pltpu.VMEM((_IBLK, CHUNK), jnp.int32),
        pltpu.VMEM((_IBLK, CHUNK), jnp.int32),
        pltpu.VMEM((2, CHUNK, ED), jnp.float32),
        pltpu.VMEM((2, CHUNK), jnp.float32),
        pltpu.VMEM_SHARED((NN, ED), jnp.float32),
        pltpu.SemaphoreType.DMA,
        pltpu.SemaphoreType.DMA,
        pltpu.SemaphoreType.DMA,
        pltpu.SemaphoreType.DMA,
        pltpu.SemaphoreType.DMA,
        pltpu.SemaphoreType.DMA,
    ],
)
def _scatter_kernel(yf_hbm, wflat_hbm, gidx_hbm, widx_hbm, didx_hbm, out_hbm,
                    gbuf, wibuf, dbuf, rows, wvals, h_sh,
                    sg0, sg1, sw0, sw1, ss0, ss1):
    cid = lax.axis_index("c")
    sid = lax.axis_index("s")
    wid = sid * NC + cid
    zero16 = jnp.zeros((16,), jnp.float32)
    sg = (sg0, sg1)
    sw = (sw0, sw1)
    ss = (ss0, ss1)

    def _issue_gather(jj, b):
        pltpu.async_copy(yf_hbm.at[gbuf.at[jj]], rows.at[b], sg[b])
        pltpu.async_copy(wflat_hbm.at[wibuf.at[jj]], wvals.at[b], sw[b])

    def _wait_gather(b):
        pltpu.make_async_copy(yf_hbm.at[gbuf.at[0]], rows.at[b], sg[b]).wait()
        pltpu.make_async_copy(wflat_hbm.at[wibuf.at[0]], wvals.at[b],
                              sw[b]).wait()

    def _issue_scatter(jj, b):
        pltpu.async_copy(rows.at[b], h_sh.at[dbuf.at[jj]], ss[b], add=True)

    def _wait_scatter(b):
        pltpu.make_async_copy(rows.at[b], h_sh.at[dbuf.at[0]], ss[b]).wait()

    def _scale(b):
        def _grp(g, c2):
            wv = wvals[b, pl.ds(g * 16, 16)]
            for i in range(16):
                wb = _bcast_lane(wv, i)
                e = g * 16 + i
                for cb in range(ED // 16):
                    sl = pl.ds(cb * 16, 16)
                    rows[b, e, sl] = rows[b, e, sl] * wb
            return c2

        lax.fori_loop(0, CHUNK // 16, _grp, 0)

    def _zrow(r, carry):
        for cb in range(ED // 16):
            rows[0, r, pl.ds(cb * 16, 16)] = zero16
        return carry

    lax.fori_loop(0, CHUNK, _zrow, 0)
    hbase = sid * HSTRIPE
    for off, sz in _SEGS:
        pltpu.sync_copy(rows.at[0, pl.ds(0, sz)],
                        h_sh.at[pl.ds(hbase + off, sz)])

    @pl.when(sid == NS - 1)
    def _zero_tail():
        pltpu.sync_copy(rows.at[0, pl.ds(0, 16)], h_sh.at[pl.ds(NN - 16, 16)])

    plsc.subcore_barrier()

    row0 = jnp.where(cid == 0, sid * _R0, NS * _R0 + sid * _R1)
    nblk = jnp.where(cid == 0, _R0 // _IBLK, _R1 // _IBLK)

    def _iblk(bi, carry):
        rb = row0 + bi * _IBLK

        @pl.when(bi >= 1)
        def _wait_prev_tail():
            _wait_scatter(1)

        pltpu.sync_copy(gidx_hbm.at[pl.ds(rb, _IBLK)], gbuf)
        pltpu.sync_copy(widx_hbm.at[pl.ds(rb, _IBLK)], wibuf)
        pltpu.sync_copy(didx_hbm.at[pl.ds(rb, _IBLK)], dbuf)
        _issue_gather(0, 0)

        def _pair(p, c1):
            @pl.when(p >= 1)
            def _w0():
                _wait_scatter(1)

            _issue_gather(2 * p + 1, 1)
            _wait_gather(0)
            _scale(0)
            _issue_scatter(2 * p, 0)
            _wait_scatter(0)

            @pl.when(p <= _IBLK // 2 - 2)
            def _pf1():
                _issue_gather(2 * p + 2, 0)

            _wait_gather(1)
            _scale(1)
            _issue_scatter(2 * p + 1, 1)
            return c1

        lax.fori_loop(0, _IBLK // 2, _pair, 0)
        return carry

    lax.fori_loop(0, nblk, _iblk, 0)
    _wait_scatter(1)
    plsc.subcore_barrier()
    for off, sz in _SEGS:
        sl = pl.ds(hbase + off, sz)
        pltpu.sync_copy(h_sh.at[sl], rows.at[0, pl.ds(0, sz)])
        pltpu.sync_copy(rows.at[0, pl.ds(0, sz)], out_hbm.at[cid, sl])

    @pl.when(sid == NS - 1)
    def _out_tail():
        sl = pl.ds(NN - 16, 16)
        pltpu.sync_copy(h_sh.at[sl], rows.at[0, pl.ds(0, 16)])
        pltpu.sync_copy(rows.at[0, pl.ds(0, 16)], out_hbm.at[cid, sl])


_BN5 = 400
_NB5 = NN // _BN5


def _comb_body(emb_ref, wemb_ref, hp_ref, out_ref):
    out_ref[...] = emb_ref[...] * wemb_ref[...] + hp_ref[0] + hp_ref[1]


def _comb_call(emb, wemb, hp):
    return pl.pallas_call(
        _comb_body,
        grid=(_NB5,),
        in_specs=[
            pl.BlockSpec((_BN5, ED), lambda i: (i, 0)),
            pl.BlockSpec((_BN5, 1), lambda i: (i, 0)),
            pl.BlockSpec((NC, _BN5, ED), lambda i: (0, i, 0)),
        ],
        out_specs=pl.BlockSpec((_BN5, ED), lambda i: (i, 0)),
        out_shape=jax.ShapeDtypeStruct((NN, ED), jnp.float32),
    )(emb, wemb, hp)


def kernel(edge_index, edge_type, embeddings, W0, b0):
    ne = edge_index.shape[1]
    t = edge_type.astype(jnp.int32)
    ei0 = edge_index[0].astype(jnp.int32)
    ei1 = edge_index[1].astype(jnp.int32)

    gidx = jnp.concatenate([t * NN + ei1, (t + NR) * NN + ei0])
    widx = jnp.concatenate([ei0 * KK + t, ei1 * KK + (t + NR)])
    didx = jnp.concatenate([ei0, ei1])
    pad = NOP - 2 * ne
    spread = lax.iota(jnp.int32, pad)
    gidx = jnp.concatenate([gidx, spread % (KK * NN)])
    widx = jnp.concatenate([widx, jnp.full((pad,), KK * NN, jnp.int32)])
    didx = jnp.concatenate([didx, spread % NN])
    gidx2 = gidx.reshape(NOPROWS, CHUNK)
    widx2 = widx.reshape(NOPROWS, CHUNK)
    didx2 = didx.reshape(NOPROWS, CHUNK)

    yf = _mm_call(embeddings, W0, b0)
    cp = _count_kernel(widx2).reshape(NC, CPAD)
    cpr = cp[:, :KK * NN].reshape(NC, NN, KK)
    wT, wemb = _wt_call(cpr)
    wflat = jnp.concatenate(
        [wT.reshape(-1), jnp.zeros((CPAD - KK * NN,), jnp.float32)])
    hp = _scatter_kernel(yf, wflat, gidx2, widx2, didx2)
    return _comb_call(embeddings, wemb, hp)


# speedup vs baseline: 21.7204x; 1.2661x over previous
import functools

import jax
import jax.numpy as jnp
from jax import lax
from jax.experimental import pallas as pl
from jax.experimental.pallas import tpu as pltpu
from jax.experimental.pallas import tpu_sc as plsc

NN = 10000
NR = 5
KK = 2 * NR
ED = 128
NC, NS, LL = 2, 16, 16
NW = NC * NS
CHUNK = 128
RPT = 160
NOP = NW * RPT * CHUNK
NOPROWS = NOP // CHUNK
CPAD = KK * NN + 96
ZR = CPAD // NS
HSTRIPE = 624
_SEGS = ((0, 128), (128, 128), (256, 128), (384, 128), (512, 112))
ORDER = (0, 5, 1, 6, 2, 7, 3, 8, 4, 9)
_IBLK = 8
_R0, _R1 = 160, 160

_mesh = plsc.VectorSubcoreMesh(core_axis_name="c", subcore_axis_name="s")


_BN1 = 400
_NB1 = NN // _BN1


def _mm_body(emb_ref, w_ref, b_ref, y_ref):
    x = emb_ref[...]
    for k in range(KK):
        y_ref[:, k * ED:(k + 1) * ED] = (
            jnp.dot(x, w_ref[k], preferred_element_type=jnp.float32)
            + b_ref[k]
        )


def _mm_call(emb, W0, b0):
    return pl.pallas_call(
        _mm_body,
        grid=(_NB1,),
        in_specs=[
            pl.BlockSpec((_BN1, ED), lambda i: (i, 0)),
            pl.BlockSpec((KK, ED, ED), lambda i: (0, 0, 0)),
            pl.BlockSpec((KK, ED), lambda i: (0, 0)),
        ],
        out_specs=pl.BlockSpec((_BN1, KK * ED), lambda i: (i, 0)),
        out_shape=jax.ShapeDtypeStruct((NN, KK * ED), jnp.float32),
    )(emb, W0, b0)


@functools.partial(
    pl.kernel,
    out_type=jax.ShapeDtypeStruct((NC * CPAD,), jnp.float32),
    mesh=_mesh,
    scratch_types=[
        pltpu.VMEM((RPT, CHUNK), jnp.int32),
        pltpu.VMEM((CHUNK,), jnp.float32),
        pltpu.VMEM((ZR,), jnp.float32),
        pltpu.VMEM_SHARED((CPAD,), jnp.float32)
    ],
)
def _count_kernel(widx_hbm, out_hbm, idxbuf, ones_v, zbuf, c_sh):
    cid = lax.axis_index("c")
    sid = lax.axis_index("s")
    wid = sid * NC + cid
    zero16 = jnp.zeros((16,), jnp.float32)
    one16 = jnp.ones((16,), jnp.float32)

    def _zb(i, carry):
        zbuf[pl.ds(i * 16, 16)] = zero16
        return carry

    lax.fori_loop(0, ZR // 16, _zb, 0)
    for i in range(CHUNK // 16):
        ones_v[pl.ds(i * 16, 16)] = one16
    pltpu.sync_copy(zbuf, c_sh.at[pl.ds(sid * ZR, ZR)])
    plsc.subcore_barrier()

    pltpu.sync_copy(widx_hbm.at[pl.ds(wid * RPT, RPT)], idxbuf)

    def _body(j, carry):
        pltpu.sync_copy(ones_v, c_sh.at[idxbuf.at[j]], add=True)
        return carry

    lax.fori_loop(0, RPT, _body, 0)
    plsc.subcore_barrier()
    pltpu.sync_copy(c_sh.at[pl.ds(sid * ZR, ZR)], zbuf)
    pltpu.sync_copy(zbuf, out_hbm.at[pl.ds(cid * CPAD + sid * ZR, ZR)])


def _wt_body(cp_ref, w_ref):
    c = cp_ref[0] + cp_ref[1]
    a = 2.0 / jnp.maximum(c, 1.0)
    rows = [None] * KK
    p = jnp.ones((1, NN), jnp.float32)
    for j in reversed(range(KK)):
        kj = ORDER[j]
        p = p * a[kj:kj + 1, :]
        rows[kj] = p
    w_ref[...] = jnp.concatenate(rows, axis=0)


def _wt_call(cpr):
    return pl.pallas_call(
        _wt_body,
        grid=(1,),
        in_specs=[pl.BlockSpec((NC, KK, NN), lambda i: (0, 0, 0))],
        out_specs=pl.BlockSpec((KK, NN), lambda i: (0, 0)),
        out_shape=jax.ShapeDtypeStruct((KK, NN), jnp.float32),
    )(cpr)


_GDN = lax.GatherDimensionNumbers(
    offset_dims=(), collapsed_slice_dims=(0,), start_index_map=(0,))


def _bcast_lane(v16, i):
    return lax.gather(
        v16, jnp.full((16, 1), i, jnp.int32), _GDN, slice_sizes=(1,),
        mode=lax.GatherScatterMode.PROMISE_IN_BOUNDS)


@functools.partial(
    pl.kernel,
    out_type=jax.ShapeDtypeStruct((NC, NN, ED), jnp.float32),
    mesh=_mesh,
    scratch_types=[
        pltpu.VMEM((_IBLK, CHUNK), jnp.int32),
        pltpu.VMEM((_IBLK, CHUNK), jnp.int32),
        pltpu.VMEM((_IBLK, CHUNK), jnp.int32),
        pltpu.VMEM((2, CHUNK, ED), jnp.float32),
        pltpu.VMEM((2, CHUNK), jnp.float32),
        pltpu.VMEM_SHARED((NN, ED), jnp.float32),
        pltpu.SemaphoreType.DMA,
        pltpu.SemaphoreType.DMA,
        pltpu.SemaphoreType.DMA,
        pltpu.SemaphoreType.DMA,
        pltpu.SemaphoreType.DMA,
        pltpu.SemaphoreType.DMA,
    ],
)
def _scatter_kernel(yf_hbm, wflat_hbm, gidx_hbm, widx_hbm, didx_hbm, out_hbm,
                    gbuf, wibuf, dbuf, rows, wvals, h_sh,
                    sg0, sg1, sw0, sw1, ss0, ss1):
    cid = lax.axis_index("c")
    sid = lax.axis_index("s")
    wid = sid * NC + cid
    zero16 = jnp.zeros((16,), jnp.float32)
    sg = (sg0, sg1)
    sw = (sw0, sw1)
    ss = (ss0, ss1)

    def _issue_gather(jj, b):
        pltpu.async_copy(yf_hbm.at[gbuf.at[jj]], rows.at[b], sg[b])
        pltpu.async_copy(wflat_hbm.at[wibuf.at[jj]], wvals.at[b], sw[b])

    def _wait_gather(b):
        pltpu.make_async_copy(yf_hbm.at[gbuf.at[0]], rows.at[b], sg[b]).wait()
        pltpu.make_async_copy(wflat_hbm.at[wibuf.at[0]], wvals.at[b],
                              sw[b]).wait()

    def _issue_scatter(jj, b):
        pltpu.async_copy(rows.at[b], h_sh.at[dbuf.at[jj]], ss[b], add=True)

    def _wait_scatter(b):
        pltpu.make_async_copy(rows.at[b], h_sh.at[dbuf.at[0]], ss[b]).wait()

    def _scale(b):
        def _grp(g, c2):
            wv = wvals[b, pl.ds(g * 16, 16)]
            for i in range(16):
                wb = _bcast_lane(wv, i)
                e = g * 16 + i
                for cb in range(ED // 16):
                    sl = pl.ds(cb * 16, 16)
                    rows[b, e, sl] = rows[b, e, sl] * wb
            return c2

        lax.fori_loop(0, CHUNK // 16, _grp, 0)

    def _zrow(r, carry):
        for cb in range(ED // 16):
            rows[0, r, pl.ds(cb * 16, 16)] = zero16
        return carry

    lax.fori_loop(0, CHUNK, _zrow, 0)
    hbase = sid * HSTRIPE
    for off, sz in _SEGS:
        pltpu.sync_copy(rows.at[0, pl.ds(0, sz)],
                        h_sh.at[pl.ds(hbase + off, sz)])

    @pl.when(sid == NS - 1)
    def _zero_tail():
        pltpu.sync_copy(rows.at[0, pl.ds(0, 16)], h_sh.at[pl.ds(NN - 16, 16)])

    plsc.subcore_barrier()

    row0 = jnp.where(cid == 0, sid * _R0, NS * _R0 + sid * _R1)
    nblk = jnp.where(cid == 0, _R0 // _IBLK, _R1 // _IBLK)

    def _iblk(bi, carry):
        rb = row0 + bi * _IBLK

        @pl.when(bi >= 1)
        def _wait_prev_tail():
            _wait_scatter(1)

        pltpu.sync_copy(gidx_hbm.at[pl.ds(rb, _IBLK)], gbuf)
        pltpu.sync_copy(widx_hbm.at[pl.ds(rb, _IBLK)], wibuf)
        pltpu.sync_copy(didx_hbm.at[pl.ds(rb, _IBLK)], dbuf)
        _issue_gather(0, 0)

        def _pair(p, c1):
            @pl.when(p >= 1)
            def _w0():
                _wait_scatter(1)

            _issue_gather(2 * p + 1, 1)
            _wait_gather(0)
            _scale(0)
            _issue_scatter(2 * p, 0)
            _wait_scatter(0)

            @pl.when(p <= _IBLK // 2 - 2)
            def _pf1():
                _issue_gather(2 * p + 2, 0)

            _wait_gather(1)
            _scale(1)
            _issue_scatter(2 * p + 1, 1)
            return c1

        lax.fori_loop(0, _IBLK // 2, _pair, 0)
        return carry

    lax.fori_loop(0, nblk, _iblk, 0)
    _wait_scatter(1)
    plsc.subcore_barrier()
    for off, sz in _SEGS:
        sl = pl.ds(hbase + off, sz)
        pltpu.sync_copy(h_sh.at[sl], rows.at[0, pl.ds(0, sz)])
        pltpu.sync_copy(rows.at[0, pl.ds(0, sz)], out_hbm.at[cid, sl])

    @pl.when(sid == NS - 1)
    def _out_tail():
        sl = pl.ds(NN - 16, 16)
        pltpu.sync_copy(h_sh.at[sl], rows.at[0, pl.ds(0, 16)])
        pltpu.sync_copy(rows.at[0, pl.ds(0, 16)], out_hbm.at[cid, sl])


_BN5 = 400
_NB5 = NN // _BN5


def _comb_body(emb_ref, wemb_ref, hp_ref, out_ref):
    out_ref[...] = emb_ref[...] * wemb_ref[...] + hp_ref[0] + hp_ref[1]


def _comb_call(emb, wemb, hp):
    return pl.pallas_call(
        _comb_body,
        grid=(_NB5,),
        in_specs=[
            pl.BlockSpec((_BN5, ED), lambda i: (i, 0)),
            pl.BlockSpec((_BN5, 1), lambda i: (i, 0)),
            pl.BlockSpec((NC, _BN5, ED), lambda i: (0, i, 0)),
        ],
        out_specs=pl.BlockSpec((_BN5, ED), lambda i: (i, 0)),
        out_shape=jax.ShapeDtypeStruct((NN, ED), jnp.float32),
    )(emb, wemb, hp)


def kernel(edge_index, edge_type, embeddings, W0, b0):
    ne = edge_index.shape[1]
    t = edge_type.astype(jnp.int32)
    ei0 = edge_index[0].astype(jnp.int32)
    ei1 = edge_index[1].astype(jnp.int32)

    gidx = jnp.concatenate([ei1 * KK + t, ei0 * KK + (t + NR)])
    widx = jnp.concatenate([t * NN + ei0, (t + NR) * NN + ei1])
    didx = jnp.concatenate([ei0, ei1])
    pad = NOP - 2 * ne
    spread = lax.iota(jnp.int32, pad)
    gidx = jnp.concatenate([gidx, spread % (KK * NN)])
    widx = jnp.concatenate([widx, jnp.full((pad,), KK * NN, jnp.int32)])
    didx = jnp.concatenate([didx, spread % NN])
    gidx2 = gidx.reshape(NOPROWS, CHUNK)
    widx2 = widx.reshape(NOPROWS, CHUNK)
    didx2 = didx.reshape(NOPROWS, CHUNK)

    yf = _mm_call(embeddings, W0, b0).reshape(NN * KK, ED)
    cp = _count_kernel(widx2).reshape(NC, CPAD)
    cpr = cp[:, :KK * NN].reshape(NC, KK, NN)
    w = _wt_call(cpr)
    wemb = w[0].reshape(NN, 1)
    wflat = jnp.concatenate(
        [w.reshape(-1), jnp.zeros((CPAD - KK * NN,), jnp.float32)])
    hp = _scatter_kernel(yf, wflat, gidx2, widx2, didx2)
    return _comb_call(embeddings, wemb, hp)


# speedup vs baseline: 22.9904x; 1.0585x over previous
import functools

import jax
import jax.numpy as jnp
from jax import lax
from jax.experimental import pallas as pl
from jax.experimental.pallas import tpu as pltpu
from jax.experimental.pallas import tpu_sc as plsc

NN = 10000
NR = 5
KK = 2 * NR
ED = 128
NC, NS, LL = 2, 16, 16
NW = NC * NS
CHUNK = 128
RPT = 160
NOP = NW * RPT * CHUNK
NOPROWS = NOP // CHUNK
CPAD = KK * NN + 96
ZR = CPAD // NS
HSTRIPE = 624
_SEGS = ((0, 128), (128, 128), (256, 128), (384, 128), (512, 112))
ORDER = (0, 5, 1, 6, 2, 7, 3, 8, 4, 9)
_IBLK = 8
_R0, _R1 = 160, 160

_mesh = plsc.VectorSubcoreMesh(core_axis_name="c", subcore_axis_name="s")


_BN1 = 400
_NB1 = NN // _BN1


def _mm_body(emb_ref, w_ref, b_ref, y_ref):
    x = emb_ref[...]
    for k in range(KK):
        y_ref[k] = (
            jnp.dot(x, w_ref[k], preferred_element_type=jnp.float32)
            + b_ref[k]
        )


def _mm_call(emb, W0, b0):
    return pl.pallas_call(
        _mm_body,
        grid=(_NB1,),
        in_specs=[
            pl.BlockSpec((_BN1, ED), lambda i: (i, 0)),
            pl.BlockSpec((KK, ED, ED), lambda i: (0, 0, 0)),
            pl.BlockSpec((KK, ED), lambda i: (0, 0)),
        ],
        out_specs=pl.BlockSpec((KK, _BN1, ED), lambda i: (0, i, 0)),
        out_shape=jax.ShapeDtypeStruct((KK, NN, ED), jnp.float32),
    )(emb, W0, b0)


@functools.partial(
    pl.kernel,
    out_type=jax.ShapeDtypeStruct((NC * CPAD,), jnp.float32),
    mesh=_mesh,
    scratch_types=[
        pltpu.VMEM((RPT, CHUNK), jnp.int32),
        pltpu.VMEM((CHUNK,), jnp.float32),
        pltpu.VMEM((ZR,), jnp.float32),
        pltpu.VMEM_SHARED((CPAD,), jnp.float32)
    ],
)
def _count_kernel(widx_hbm, out_hbm, idxbuf, ones_v, zbuf, c_sh):
    cid = lax.axis_index("c")
    sid = lax.axis_index("s")
    wid = sid * NC + cid
    zero16 = jnp.zeros((16,), jnp.float32)
    one16 = jnp.ones((16,), jnp.float32)

    def _zb(i, carry):
        zbuf[pl.ds(i * 16, 16)] = zero16
        return carry

    lax.fori_loop(0, ZR // 16, _zb, 0)
    for i in range(CHUNK // 16):
        ones_v[pl.ds(i * 16, 16)] = one16
    pltpu.sync_copy(zbuf, c_sh.at[pl.ds(sid * ZR, ZR)])
    plsc.subcore_barrier()

    pltpu.sync_copy(widx_hbm.at[pl.ds(wid * RPT, RPT)], idxbuf)

    def _body(j, carry):
        pltpu.sync_copy(ones_v, c_sh.at[idxbuf.at[j]], add=True)
        return carry

    lax.fori_loop(0, RPT, _body, 0)
    plsc.subcore_barrier()
    pltpu.sync_copy(c_sh.at[pl.ds(sid * ZR, ZR)], zbuf)
    pltpu.sync_copy(zbuf, out_hbm.at[pl.ds(cid * CPAD + sid * ZR, ZR)])


def _wt_body(cp_ref, w_ref):
    c = cp_ref[0] + cp_ref[1]
    a = 2.0 / jnp.maximum(c, 1.0)
    rows = [None] * KK
    p = jnp.ones((1, NN), jnp.float32)
    for j in reversed(range(KK)):
        kj = ORDER[j]
        p = p * a[kj:kj + 1, :]
        rows[kj] = p
    w_ref[...] = jnp.concatenate(rows, axis=0)


def _wt_call(cpr):
    return pl.pallas_call(
        _wt_body,
        grid=(1,),
        in_specs=[pl.BlockSpec((NC, KK, NN), lambda i: (0, 0, 0))],
        out_specs=pl.BlockSpec((KK, NN), lambda i: (0, 0)),
        out_shape=jax.ShapeDtypeStruct((KK, NN), jnp.float32),
    )(cpr)


_GDN = lax.GatherDimensionNumbers(
    offset_dims=(), collapsed_slice_dims=(0,), start_index_map=(0,))


def _bcast_lane(v16, i):
    return lax.gather(
        v16, jnp.full((16, 1), i, jnp.int32), _GDN, slice_sizes=(1,),
        mode=lax.GatherScatterMode.PROMISE_IN_BOUNDS)


@functools.partial(
    pl.kernel,
    out_type=jax.ShapeDtypeStruct((NC, NN, ED), jnp.float32),
    mesh=_mesh,
    scratch_types=[
        pltpu.VMEM((_IBLK, CHUNK), jnp.int32),
        pltpu.VMEM((_IBLK, CHUNK), jnp.int32),
        pltpu.VMEM((_IBLK * 2, CHUNK // 2), jnp.int32),
        pltpu.VMEM((2, CHUNK, ED), jnp.float32),
        pltpu.VMEM((2, CHUNK), jnp.float32),
        pltpu.VMEM_SHARED((NN, ED), jnp.float32),
        pltpu.SemaphoreType.DMA,
        pltpu.SemaphoreType.DMA,
        pltpu.SemaphoreType.DMA,
        pltpu.SemaphoreType.DMA,
        pltpu.SemaphoreType.DMA,
        pltpu.SemaphoreType.DMA,
    ],
)
def _scatter_kernel(yf_hbm, wflat_hbm, gidx_hbm, widx_hbm, didx_hbm, out_hbm,
                    gbuf, wibuf, dbuf, rows, wvals, h_sh,
                    sg0, sg1, sw0, sw1, ss0, ss1):
    cid = lax.axis_index("c")
    sid = lax.axis_index("s")
    wid = sid * NC + cid
    zero16 = jnp.zeros((16,), jnp.float32)
    sg = (sg0, sg1)
    sw = (sw0, sw1)
    ss = (ss0, ss1)

    def _issue_gather(jj, b):
        pltpu.async_copy(yf_hbm.at[gbuf.at[jj]], rows.at[b], sg[b])
        pltpu.async_copy(wflat_hbm.at[wibuf.at[jj]], wvals.at[b], sw[b])

    def _wait_gather(b):
        pltpu.make_async_copy(yf_hbm.at[gbuf.at[0]], rows.at[b], sg[b]).wait()
        pltpu.make_async_copy(wflat_hbm.at[wibuf.at[0]], wvals.at[b],
                              sw[b]).wait()

    def _issue_scatter_half(r2, b, h):
        pltpu.async_copy(rows.at[b, pl.ds(h * (CHUNK // 2), CHUNK // 2)],
                         h_sh.at[dbuf.at[r2]], ss[b], add=True)

    def _wait_scatter(b):
        for h in (0, 1):
            pltpu.make_async_copy(
                rows.at[b, pl.ds(h * (CHUNK // 2), CHUNK // 2)],
                h_sh.at[dbuf.at[0]], ss[b]).wait()

    def _scale_half(b, h):
        def _grp(g, c2):
            g2 = g + h * (CHUNK // 32)
            wv = wvals[b, pl.ds(g2 * 16, 16)]
            for i in range(16):
                wb = _bcast_lane(wv, i)
                e = g2 * 16 + i
                for cb in range(ED // 16):
                    sl = pl.ds(cb * 16, 16)
                    rows[b, e, sl] = rows[b, e, sl] * wb
            return c2

        lax.fori_loop(0, CHUNK // 32, _grp, 0)

    def _zrow(r, carry):
        for cb in range(ED // 16):
            rows[0, r, pl.ds(cb * 16, 16)] = zero16
        return carry

    lax.fori_loop(0, CHUNK, _zrow, 0)
    hbase = sid * HSTRIPE
    for off, sz in _SEGS:
        pltpu.sync_copy(rows.at[0, pl.ds(0, sz)],
                        h_sh.at[pl.ds(hbase + off, sz)])

    @pl.when(sid == NS - 1)
    def _zero_tail():
        pltpu.sync_copy(rows.at[0, pl.ds(0, 16)], h_sh.at[pl.ds(NN - 16, 16)])

    plsc.subcore_barrier()

    row0 = jnp.where(cid == 0, sid * _R0, NS * _R0 + sid * _R1)
    nblk = jnp.where(cid == 0, _R0 // _IBLK, _R1 // _IBLK)

    def _iblk(bi, carry):
        rb = row0 + bi * _IBLK

        @pl.when(bi >= 1)
        def _wait_prev_tail():
            _wait_scatter(1)

        pltpu.sync_copy(gidx_hbm.at[pl.ds(rb, _IBLK)], gbuf)
        pltpu.sync_copy(widx_hbm.at[pl.ds(rb, _IBLK)], wibuf)
        pltpu.sync_copy(didx_hbm.at[pl.ds(rb * 2, _IBLK * 2)], dbuf)
        _issue_gather(0, 0)

        def _pair(p, c1):
            _wait_gather(0)
            _scale_half(0, 0)
            _issue_scatter_half(4 * p, 0, 0)

            @pl.when(p >= 1)
            def _w1():
                _wait_scatter(1)

            _issue_gather(2 * p + 1, 1)
            _scale_half(0, 1)
            _issue_scatter_half(4 * p + 1, 0, 1)
            _wait_gather(1)
            _scale_half(1, 0)
            _issue_scatter_half(4 * p + 2, 1, 0)
            _wait_scatter(0)

            @pl.when(p <= _IBLK // 2 - 2)
            def _pf1():
                _issue_gather(2 * p + 2, 0)

            _scale_half(1, 1)
            _issue_scatter_half(4 * p + 3, 1, 1)
            return c1

        lax.fori_loop(0, _IBLK // 2, _pair, 0)
        return carry

    lax.fori_loop(0, nblk, _iblk, 0)
    _wait_scatter(1)
    plsc.subcore_barrier()
    for off, sz in _SEGS:
        sl = pl.ds(hbase + off, sz)
        pltpu.sync_copy(h_sh.at[sl], rows.at[0, pl.ds(0, sz)])
        pltpu.sync_copy(rows.at[0, pl.ds(0, sz)], out_hbm.at[cid, sl])

    @pl.when(sid == NS - 1)
    def _out_tail():
        sl = pl.ds(NN - 16, 16)
        pltpu.sync_copy(h_sh.at[sl], rows.at[0, pl.ds(0, 16)])
        pltpu.sync_copy(rows.at[0, pl.ds(0, 16)], out_hbm.at[cid, sl])


_BN5 = 400
_NB5 = NN // _BN5


def _comb_body(emb_ref, wemb_ref, hp_ref, out_ref):
    out_ref[...] = emb_ref[...] * wemb_ref[...] + hp_ref[0] + hp_ref[1]


def _comb_call(emb, wemb, hp):
    return pl.pallas_call(
        _comb_body,
        grid=(_NB5,),
        in_specs=[
            pl.BlockSpec((_BN5, ED), lambda i: (i, 0)),
            pl.BlockSpec((_BN5, 1), lambda i: (i, 0)),
            pl.BlockSpec((NC, _BN5, ED), lambda i: (0, i, 0)),
        ],
        out_specs=pl.BlockSpec((_BN5, ED), lambda i: (i, 0)),
        out_shape=jax.ShapeDtypeStruct((NN, ED), jnp.float32),
    )(emb, wemb, hp)


def kernel(edge_index, edge_type, embeddings, W0, b0):
    ne = edge_index.shape[1]
    t = edge_type.astype(jnp.int32)
    ei0 = edge_index[0].astype(jnp.int32)
    ei1 = edge_index[1].astype(jnp.int32)

    gidx = jnp.concatenate([t * NN + ei1, (t + NR) * NN + ei0])
    widx = jnp.concatenate([t * NN + ei0, (t + NR) * NN + ei1])
    didx = jnp.concatenate([ei0, ei1])
    pad = NOP - 2 * ne
    spread = lax.iota(jnp.int32, pad)
    gidx = jnp.concatenate([gidx, spread % (KK * NN)])
    widx = jnp.concatenate([widx, jnp.full((pad,), KK * NN, jnp.int32)])
    didx = jnp.concatenate([didx, spread % NN])
    gidx2 = gidx.reshape(NOPROWS, CHUNK)
    widx2 = widx.reshape(NOPROWS, CHUNK)
    didx2 = didx.reshape(NOPROWS * 2, CHUNK // 2)

    yf = _mm_call(embeddings, W0, b0).reshape(KK * NN, ED)
    cp = _count_kernel(widx2).reshape(NC, CPAD)
    cpr = cp[:, :KK * NN].reshape(NC, KK, NN)
    w = _wt_call(cpr)
    wemb = w[0].reshape(NN, 1)
    wflat = jnp.concatenate(
        [w.reshape(-1), jnp.zeros((CPAD - KK * NN,), jnp.float32)])
    hp = _scatter_kernel(yf, wflat, gidx2, widx2, didx2)
    return _comb_call(embeddings, wemb, hp)


# speedup vs baseline: 24.3319x; 1.0584x over previous
import functools

import jax
import jax.numpy as jnp
from jax import lax
from jax.experimental import pallas as pl
from jax.experimental.pallas import tpu as pltpu
from jax.experimental.pallas import tpu_sc as plsc

NN = 10000
NR = 5
KK = 2 * NR
ED = 128
NC, NS, LL = 2, 16, 16
NW = NC * NS
CHUNK = 128
RPT = 160
NOP = NW * RPT * CHUNK
NOPROWS = NOP // CHUNK
CPAD = KK * NN + 96
ZR = CPAD // NS
HSTRIPE = 624
_SEGS = ((0, 128), (128, 128), (256, 128), (384, 128), (512, 112))
ORDER = (0, 5, 1, 6, 2, 7, 3, 8, 4, 9)
_IBLK = 8
_R0, _R1 = 160, 160

_mesh = plsc.VectorSubcoreMesh(core_axis_name="c", subcore_axis_name="s")


_BN1 = 400
_NB1 = NN // _BN1


def _mm_body(emb_ref, w_ref, b_ref, y_ref):
    x = emb_ref[...]
    for k in range(KK):
        y_ref[k] = (
            jnp.dot(x, w_ref[k], preferred_element_type=jnp.float32)
            + b_ref[k]
        )


def _mm_call(emb, W0, b0):
    return pl.pallas_call(
        _mm_body,
        grid=(_NB1,),
        in_specs=[
            pl.BlockSpec((_BN1, ED), lambda i: (i, 0)),
            pl.BlockSpec((KK, ED, ED), lambda i: (0, 0, 0)),
            pl.BlockSpec((KK, ED), lambda i: (0, 0)),
        ],
        out_specs=pl.BlockSpec((KK, _BN1, ED), lambda i: (0, i, 0)),
        out_shape=jax.ShapeDtypeStruct((KK, NN, ED), jnp.float32),
    )(emb, W0, b0)


@functools.partial(
    pl.kernel,
    out_type=jax.ShapeDtypeStruct((NC * CPAD,), jnp.float32),
    mesh=_mesh,
    scratch_types=[
        pltpu.VMEM((RPT, CHUNK), jnp.int32),
        pltpu.VMEM((CHUNK,), jnp.float32),
        pltpu.VMEM((ZR,), jnp.float32),
        pltpu.VMEM_SHARED((CPAD,), jnp.float32)
    ],
)
def _count_kernel(widx_hbm, out_hbm, idxbuf, ones_v, zbuf, c_sh):
    cid = lax.axis_index("c")
    sid = lax.axis_index("s")
    wid = sid * NC + cid
    zero16 = jnp.zeros((16,), jnp.float32)
    one16 = jnp.ones((16,), jnp.float32)

    def _zb(i, carry):
        zbuf[pl.ds(i * 16, 16)] = zero16
        return carry

    lax.fori_loop(0, ZR // 16, _zb, 0)
    for i in range(CHUNK // 16):
        ones_v[pl.ds(i * 16, 16)] = one16
    pltpu.sync_copy(zbuf, c_sh.at[pl.ds(sid * ZR, ZR)])
    plsc.subcore_barrier()

    pltpu.sync_copy(widx_hbm.at[pl.ds(wid * RPT, RPT)], idxbuf)

    def _body(j, carry):
        pltpu.sync_copy(ones_v, c_sh.at[idxbuf.at[j]], add=True)
        return carry

    lax.fori_loop(0, RPT, _body, 0)
    plsc.subcore_barrier()
    pltpu.sync_copy(c_sh.at[pl.ds(sid * ZR, ZR)], zbuf)
    pltpu.sync_copy(zbuf, out_hbm.at[pl.ds(cid * CPAD + sid * ZR, ZR)])


def _wt_body(cp_ref, w_ref):
    c = cp_ref[0] + cp_ref[1]
    a = 2.0 / jnp.maximum(c, 1.0)
    rows = [None] * KK
    p = jnp.ones((1, NN), jnp.float32)
    for j in reversed(range(KK)):
        kj = ORDER[j]
        p = p * a[kj:kj + 1, :]
        rows[kj] = p
    w_ref[...] = jnp.concatenate(rows, axis=0)


def _wt_call(cpr):
    return pl.pallas_call(
        _wt_body,
        grid=(1,),
        in_specs=[pl.BlockSpec((NC, KK, NN), lambda i: (0, 0, 0))],
        out_specs=pl.BlockSpec((KK, NN), lambda i: (0, 0)),
        out_shape=jax.ShapeDtypeStruct((KK, NN), jnp.float32),
    )(cpr)


_GDN = lax.GatherDimensionNumbers(
    offset_dims=(), collapsed_slice_dims=(0,), start_index_map=(0,))


def _bcast_lane(v16, i):
    return lax.gather(
        v16, jnp.full((16, 1), i, jnp.int32), _GDN, slice_sizes=(1,),
        mode=lax.GatherScatterMode.PROMISE_IN_BOUNDS)


@functools.partial(
    pl.kernel,
    out_type=jax.ShapeDtypeStruct((NC, NN, ED), jnp.float32),
    mesh=_mesh,
    scratch_types=[
        pltpu.VMEM((_IBLK, CHUNK), jnp.int32),
        pltpu.VMEM((_IBLK, CHUNK), jnp.int32),
        pltpu.VMEM((_IBLK, CHUNK), jnp.int32),
        pltpu.VMEM((2, CHUNK, ED), jnp.float32),
        pltpu.VMEM((2, CHUNK), jnp.float32),
        pltpu.VMEM_SHARED((NN, ED), jnp.float32),
        pltpu.SemaphoreType.DMA,
        pltpu.SemaphoreType.DMA,
        pltpu.SemaphoreType.DMA,
        pltpu.SemaphoreType.DMA,
        pltpu.SemaphoreType.DMA,
        pltpu.SemaphoreType.DMA,
    ],
)
def _scatter_kernel(yf_hbm, wflat_hbm, gidx_hbm, widx_hbm, didx_hbm, out_hbm,
                    gbuf, wibuf, dbuf, rows, wvals, h_sh,
                    sg0, sg1, sw0, sw1, ss0, ss1):
    cid = lax.axis_index("c")
    sid = lax.axis_index("s")
    wid = sid * NC + cid
    zero16 = jnp.zeros((16,), jnp.float32)
    sg = (sg0, sg1)
    sw = (sw0, sw1)
    ss = (ss0, ss1)

    def _issue_gather(jj, b):
        pltpu.async_copy(yf_hbm.at[gbuf.at[jj]], rows.at[b], sg[b])
        pltpu.async_copy(wflat_hbm.at[wibuf.at[jj]], wvals.at[b], sw[b])

    def _wait_gather(b):
        pltpu.make_async_copy(yf_hbm.at[gbuf.at[0]], rows.at[b], sg[b]).wait()
        pltpu.make_async_copy(wflat_hbm.at[wibuf.at[0]], wvals.at[b],
                              sw[b]).wait()

    def _issue_scatter(jj, b):
        pltpu.async_copy(rows.at[b], h_sh.at[dbuf.at[jj]], ss[b], add=True)

    def _wait_scatter(b):
        pltpu.make_async_copy(rows.at[b], h_sh.at[dbuf.at[0]], ss[b]).wait()

    def _scale(b):
        def _grp(g, c2):
            wv = wvals[b, pl.ds(g * 16, 16)]
            for i in range(16):
                wb = _bcast_lane(wv, i)
                e = g * 16 + i
                for cb in range(ED // 16):
                    sl = pl.ds(cb * 16, 16)
                    rows[b, e, sl] = rows[b, e, sl] * wb
            return c2

        lax.fori_loop(0, CHUNK // 16, _grp, 0)

    def _zrow(r, carry):
        for cb in range(ED // 16):
            rows[0, r, pl.ds(cb * 16, 16)] = zero16
        return carry

    lax.fori_loop(0, CHUNK, _zrow, 0)
    hbase = sid * HSTRIPE
    for off, sz in _SEGS:
        pltpu.sync_copy(rows.at[0, pl.ds(0, sz)],
                        h_sh.at[pl.ds(hbase + off, sz)])

    @pl.when(sid == NS - 1)
    def _zero_tail():
        pltpu.sync_copy(rows.at[0, pl.ds(0, 16)], h_sh.at[pl.ds(NN - 16, 16)])

    plsc.subcore_barrier()

    row0 = jnp.where(cid == 0, sid * _R0, NS * _R0 + sid * _R1)
    nblk = jnp.where(cid == 0, _R0 // _IBLK, _R1 // _IBLK)

    def _iblk(bi, carry):
        rb = row0 + bi * _IBLK

        @pl.when(bi >= 1)
        def _wait_prev_tail():
            _wait_scatter(1)

        pltpu.sync_copy(gidx_hbm.at[pl.ds(rb, _IBLK)], gbuf)
        pltpu.sync_copy(widx_hbm.at[pl.ds(rb, _IBLK)], wibuf)
        pltpu.sync_copy(didx_hbm.at[pl.ds(rb, _IBLK)], dbuf)
        _issue_gather(0, 0)

        def _pair(p, c1):
            @pl.when(p >= 1)
            def _w0():
                _wait_scatter(1)

            _issue_gather(2 * p + 1, 1)
            _wait_gather(0)
            _scale(0)
            _issue_scatter(2 * p, 0)
            _wait_scatter(0)

            @pl.when(p <= _IBLK // 2 - 2)
            def _pf1():
                _issue_gather(2 * p + 2, 0)

            _wait_gather(1)
            _scale(1)
            _issue_scatter(2 * p + 1, 1)
            return c1

        lax.fori_loop(0, _IBLK // 2, _pair, 0)
        return carry

    lax.fori_loop(0, nblk, _iblk, 0)
    _wait_scatter(1)
    plsc.subcore_barrier()
    for off, sz in _SEGS:
        sl = pl.ds(hbase + off, sz)
        pltpu.sync_copy(h_sh.at[sl], rows.at[0, pl.ds(0, sz)])
        pltpu.sync_copy(rows.at[0, pl.ds(0, sz)], out_hbm.at[cid, sl])

    @pl.when(sid == NS - 1)
    def _out_tail():
        sl = pl.ds(NN - 16, 16)
        pltpu.sync_copy(h_sh.at[sl], rows.at[0, pl.ds(0, 16)])
        pltpu.sync_copy(rows.at[0, pl.ds(0, 16)], out_hbm.at[cid, sl])


_BN5 = 400
_NB5 = NN // _BN5


def _comb_body(emb_ref, wemb_ref, hp_ref, out_ref):
    out_ref[...] = emb_ref[...] * wemb_ref[...] + hp_ref[0] + hp_ref[1]


def _comb_call(emb, wemb, hp):
    return pl.pallas_call(
        _comb_body,
        grid=(_NB5,),
        in_specs=[
            pl.BlockSpec((_BN5, ED), lambda i: (i, 0)),
            pl.BlockSpec((_BN5, 1), lambda i: (i, 0)),
            pl.BlockSpec((NC, _BN5, ED), lambda i: (0, i, 0)),
        ],
        out_specs=pl.BlockSpec((_BN5, ED), lambda i: (i, 0)),
        out_shape=jax.ShapeDtypeStruct((NN, ED), jnp.float32),
    )(emb, wemb, hp)


def kernel(edge_index, edge_type, embeddings, W0, b0):
    ne = edge_index.shape[1]
    t = edge_type.astype(jnp.int32)
    ei0 = edge_index[0].astype(jnp.int32)
    ei1 = edge_index[1].astype(jnp.int32)

    gidx = jnp.concatenate([t * NN + ei1, (t + NR) * NN + ei0])
    widx = jnp.concatenate([t * NN + ei0, (t + NR) * NN + ei1])
    didx = jnp.concatenate([ei0, ei1])
    pad = NOP - 2 * ne
    spread = lax.iota(jnp.int32, pad)
    gidx = jnp.concatenate([gidx, spread % (KK * NN)])
    widx = jnp.concatenate([widx, jnp.full((pad,), KK * NN, jnp.int32)])
    didx = jnp.concatenate([didx, spread % NN])
    gidx2 = gidx.reshape(NOPROWS, CHUNK)
    widx2 = widx.reshape(NOPROWS, CHUNK)
    didx2 = didx.reshape(NOPROWS, CHUNK)

    yf = _mm_call(embeddings, W0, b0).reshape(KK * NN, ED)
    cp = _count_kernel(widx2).reshape(NC, CPAD)
    cpr = cp[:, :KK * NN].reshape(NC, KK, NN)
    w = _wt_call(cpr)
    wemb = w[0].reshape(NN, 1)
    wflat = jnp.concatenate(
        [w.reshape(-1), jnp.zeros((CPAD - KK * NN,), jnp.float32)])
    hp = _scatter_kernel(yf, wflat, gidx2, widx2, didx2)
    return _comb_call(embeddings, wemb, hp)


# speedup vs baseline: 26.4639x; 1.0876x over previous
import functools

import jax
import jax.numpy as jnp
from jax import lax
from jax.experimental import pallas as pl
from jax.experimental.pallas import tpu as pltpu
from jax.experimental.pallas import tpu_sc as plsc

NN = 10000
NR = 5
KK = 2 * NR
ED = 128
NC, NS, LL = 2, 16, 16
NW = NC * NS
CHUNK = 128
RPT = 160
NOP = NW * RPT * CHUNK
NOPROWS = NOP // CHUNK
CPAD = KK * NN + 96
ZR = CPAD // NS
HSTRIPE = 624
_SEGS = ((0, 128), (128, 128), (256, 128), (384, 128), (512, 112))
ORDER = (0, 5, 1, 6, 2, 7, 3, 8, 4, 9)
_IBLK = 16
_R0, _R1 = 160, 160

_mesh = plsc.VectorSubcoreMesh(core_axis_name="c", subcore_axis_name="s")


_BN1 = 400
_NB1 = NN // _BN1


def _mm_body(emb_ref, w_ref, b_ref, y_ref):
    x = emb_ref[...]
    for k in range(KK):
        y_ref[k] = (
            jnp.dot(x, w_ref[k], preferred_element_type=jnp.float32)
            + b_ref[k]
        )


def _mm_call(emb, W0, b0):
    return pl.pallas_call(
        _mm_body,
        grid=(_NB1,),
        in_specs=[
            pl.BlockSpec((_BN1, ED), lambda i: (i, 0)),
            pl.BlockSpec((KK, ED, ED), lambda i: (0, 0, 0)),
            pl.BlockSpec((KK, ED), lambda i: (0, 0)),
        ],
        out_specs=pl.BlockSpec((KK, _BN1, ED), lambda i: (0, i, 0)),
        out_shape=jax.ShapeDtypeStruct((KK, NN, ED), jnp.float32),
    )(emb, W0, b0)


@functools.partial(
    pl.kernel,
    out_type=jax.ShapeDtypeStruct((NC * CPAD,), jnp.float32),
    mesh=_mesh,
    scratch_types=[
        pltpu.VMEM((RPT, CHUNK), jnp.int32),
        pltpu.VMEM((CHUNK,), jnp.float32),
        pltpu.VMEM((ZR,), jnp.float32),
        pltpu.VMEM_SHARED((CPAD,), jnp.float32)
    ],
)
def _count_kernel(widx_hbm, out_hbm, idxbuf, ones_v, zbuf, c_sh):
    cid = lax.axis_index("c")
    sid = lax.axis_index("s")
    wid = sid * NC + cid
    zero16 = jnp.zeros((16,), jnp.float32)
    one16 = jnp.ones((16,), jnp.float32)

    def _zb(i, carry):
        zbuf[pl.ds(i * 16, 16)] = zero16
        return carry

    lax.fori_loop(0, ZR // 16, _zb, 0)
    for i in range(CHUNK // 16):
        ones_v[pl.ds(i * 16, 16)] = one16
    pltpu.sync_copy(zbuf, c_sh.at[pl.ds(sid * ZR, ZR)])
    plsc.subcore_barrier()

    pltpu.sync_copy(widx_hbm.at[pl.ds(wid * RPT, RPT)], idxbuf)

    def _body(j, carry):
        pltpu.sync_copy(ones_v, c_sh.at[idxbuf.at[j]], add=True)
        return carry

    lax.fori_loop(0, RPT, _body, 0)
    plsc.subcore_barrier()
    pltpu.sync_copy(c_sh.at[pl.ds(sid * ZR, ZR)], zbuf)
    pltpu.sync_copy(zbuf, out_hbm.at[pl.ds(cid * CPAD + sid * ZR, ZR)])


def _wt_body(cp_ref, w_ref):
    c = cp_ref[0] + cp_ref[1]
    a = 2.0 / jnp.maximum(c, 1.0)
    rows = [None] * KK
    p = jnp.ones((1, NN), jnp.float32)
    for j in reversed(range(KK)):
        kj = ORDER[j]
        p = p * a[kj:kj + 1, :]
        rows[kj] = p
    w_ref[...] = jnp.concatenate(rows, axis=0)


def _wt_call(cpr):
    return pl.pallas_call(
        _wt_body,
        grid=(1,),
        in_specs=[pl.BlockSpec((NC, KK, NN), lambda i: (0, 0, 0))],
        out_specs=pl.BlockSpec((KK, NN), lambda i: (0, 0)),
        out_shape=jax.ShapeDtypeStruct((KK, NN), jnp.float32),
    )(cpr)


_GDN = lax.GatherDimensionNumbers(
    offset_dims=(), collapsed_slice_dims=(0,), start_index_map=(0,))


def _bcast_lane(v16, i):
    return lax.gather(
        v16, jnp.full((16, 1), i, jnp.int32), _GDN, slice_sizes=(1,),
        mode=lax.GatherScatterMode.PROMISE_IN_BOUNDS)


@functools.partial(
    pl.kernel,
    out_type=jax.ShapeDtypeStruct((NC, NN, ED), jnp.float32),
    mesh=_mesh,
    scratch_types=[
        pltpu.VMEM((_IBLK, CHUNK), jnp.int32),
        pltpu.VMEM((_IBLK, CHUNK), jnp.int32),
        pltpu.VMEM((_IBLK, CHUNK), jnp.int32),
        pltpu.VMEM((2, CHUNK, ED), jnp.float32),
        pltpu.VMEM((2, CHUNK), jnp.float32),
        pltpu.VMEM_SHARED((NN, ED), jnp.float32),
        pltpu.SemaphoreType.DMA,
        pltpu.SemaphoreType.DMA,
        pltpu.SemaphoreType.DMA,
        pltpu.SemaphoreType.DMA,
        pltpu.SemaphoreType.DMA,
        pltpu.SemaphoreType.DMA,
    ],
)
def _scatter_kernel(yf_hbm, wflat_hbm, gidx_hbm, widx_hbm, didx_hbm, out_hbm,
                    gbuf, wibuf, dbuf, rows, wvals, h_sh,
                    sg0, sg1, sw0, sw1, ss0, ss1):
    cid = lax.axis_index("c")
    sid = lax.axis_index("s")
    wid = sid * NC + cid
    zero16 = jnp.zeros((16,), jnp.float32)
    sg = (sg0, sg1)
    sw = (sw0, sw1)
    ss = (ss0, ss1)

    def _issue_gather(jj, b):
        pltpu.async_copy(yf_hbm.at[gbuf.at[jj]], rows.at[b], sg[b])
        pltpu.async_copy(wflat_hbm.at[wibuf.at[jj]], wvals.at[b], sw[b])

    def _wait_gather(b):
        pltpu.make_async_copy(yf_hbm.at[gbuf.at[0]], rows.at[b], sg[b]).wait()
        pltpu.make_async_copy(wflat_hbm.at[wibuf.at[0]], wvals.at[b],
                              sw[b]).wait()

    def _issue_scatter(jj, b):
        pltpu.async_copy(rows.at[b], h_sh.at[dbuf.at[jj]], ss[b], add=True)

    def _wait_scatter(b):
        pltpu.make_async_copy(rows.at[b], h_sh.at[dbuf.at[0]], ss[b]).wait()

    def _scale(b):
        def _grp(g, c2):
            wv = wvals[b, pl.ds(g * 16, 16)]
            for i in range(16):
                wb = _bcast_lane(wv, i)
                e = g * 16 + i
                for cb in range(ED // 16):
                    sl = pl.ds(cb * 16, 16)
                    rows[b, e, sl] = rows[b, e, sl] * wb
            return c2

        lax.fori_loop(0, CHUNK // 16, _grp, 0)

    def _zrow(r, carry):
        for cb in range(ED // 16):
            rows[0, r, pl.ds(cb * 16, 16)] = zero16
        return carry

    lax.fori_loop(0, CHUNK, _zrow, 0)
    hbase = sid * HSTRIPE
    for off, sz in _SEGS:
        pltpu.sync_copy(rows.at[0, pl.ds(0, sz)],
                        h_sh.at[pl.ds(hbase + off, sz)])

    @pl.when(sid == NS - 1)
    def _zero_tail():
        pltpu.sync_copy(rows.at[0, pl.ds(0, 16)], h_sh.at[pl.ds(NN - 16, 16)])

    plsc.subcore_barrier()

    row0 = jnp.where(cid == 0, sid * _R0, NS * _R0 + sid * _R1)
    nblk = jnp.where(cid == 0, _R0 // _IBLK, _R1 // _IBLK)

    def _iblk(bi, carry):
        rb = row0 + bi * _IBLK

        @pl.when(bi >= 1)
        def _wait_prev_tail():
            _wait_scatter(1)

        pltpu.sync_copy(gidx_hbm.at[pl.ds(rb, _IBLK)], gbuf)
        pltpu.sync_copy(widx_hbm.at[pl.ds(rb, _IBLK)], wibuf)
        pltpu.sync_copy(didx_hbm.at[pl.ds(rb, _IBLK)], dbuf)
        _issue_gather(0, 0)

        def _pair(p, c1):
            @pl.when(p >= 1)
            def _w0():
                _wait_scatter(1)

            _issue_gather(2 * p + 1, 1)
            _wait_gather(0)
            _scale(0)
            _issue_scatter(2 * p, 0)
            _wait_scatter(0)

            @pl.when(p <= _IBLK // 2 - 2)
            def _pf1():
                _issue_gather(2 * p + 2, 0)

            _wait_gather(1)
            _scale(1)
            _issue_scatter(2 * p + 1, 1)
            return c1

        lax.fori_loop(0, _IBLK // 2, _pair, 0)
        return carry

    lax.fori_loop(0, nblk, _iblk, 0)
    _wait_scatter(1)
    plsc.subcore_barrier()
    for off, sz in _SEGS:
        sl = pl.ds(hbase + off, sz)
        pltpu.sync_copy(h_sh.at[sl], rows.at[0, pl.ds(0, sz)])
        pltpu.sync_copy(rows.at[0, pl.ds(0, sz)], out_hbm.at[cid, sl])

    @pl.when(sid == NS - 1)
    def _out_tail():
        sl = pl.ds(NN - 16, 16)
        pltpu.sync_copy(h_sh.at[sl], rows.at[0, pl.ds(0, 16)])
        pltpu.sync_copy(rows.at[0, pl.ds(0, 16)], out_hbm.at[cid, sl])


_BN5 = 400
_NB5 = NN // _BN5


def _comb_body(emb_ref, wemb_ref, hp_ref, out_ref):
    out_ref[...] = emb_ref[...] * wemb_ref[...] + hp_ref[0] + hp_ref[1]


def _comb_call(emb, wemb, hp):
    return pl.pallas_call(
        _comb_body,
        grid=(_NB5,),
        in_specs=[
            pl.BlockSpec((_BN5, ED), lambda i: (i, 0)),
            pl.BlockSpec((_BN5, 1), lambda i: (i, 0)),
            pl.BlockSpec((NC, _BN5, ED), lambda i: (0, i, 0)),
        ],
        out_specs=pl.BlockSpec((_BN5, ED), lambda i: (i, 0)),
        out_shape=jax.ShapeDtypeStruct((NN, ED), jnp.float32),
    )(emb, wemb, hp)


def kernel(edge_index, edge_type, embeddings, W0, b0):
    ne = edge_index.shape[1]
    t = edge_type.astype(jnp.int32)
    ei0 = edge_index[0].astype(jnp.int32)
    ei1 = edge_index[1].astype(jnp.int32)

    gidx = jnp.concatenate([t * NN + ei1, (t + NR) * NN + ei0])
    widx = jnp.concatenate([t * NN + ei0, (t + NR) * NN + ei1])
    didx = jnp.concatenate([ei0, ei1])
    pad = NOP - 2 * ne
    spread = lax.iota(jnp.int32, pad)
    gidx = jnp.concatenate([gidx, spread % (KK * NN)])
    widx = jnp.concatenate([widx, jnp.full((pad,), KK * NN, jnp.int32)])
    didx = jnp.concatenate([didx, spread % NN])
    gidx2 = gidx.reshape(NOPROWS, CHUNK)
    widx2 = widx.reshape(NOPROWS, CHUNK)
    didx2 = didx.reshape(NOPROWS, CHUNK)

    yf = _mm_call(embeddings, W0, b0).reshape(KK * NN, ED)
    cp = _count_kernel(widx2).reshape(NC, CPAD)
    cpr = cp[:, :KK * NN].reshape(NC, KK, NN)
    w = _wt_call(cpr)
    wemb = w[0].reshape(NN, 1)
    wflat = jnp.concatenate(
        [w.reshape(-1), jnp.zeros((CPAD - KK * NN,), jnp.float32)])
    hp = _scatter_kernel(yf, wflat, gidx2, widx2, didx2)
    return _comb_call(embeddings, wemb, hp)


# speedup vs baseline: 27.7549x; 1.0488x over previous
import functools

import jax
import jax.numpy as jnp
from jax import lax
from jax.experimental import pallas as pl
from jax.experimental.pallas import tpu as pltpu
from jax.experimental.pallas import tpu_sc as plsc

NN = 10000
NR = 5
KK = 2 * NR
ED = 128
NC, NS, LL = 2, 16, 16
NW = NC * NS
CHUNK = 128
RPT = 160
NOP = NW * RPT * CHUNK
NOPROWS = NOP // CHUNK
CPAD = KK * NN + 96
ZR = CPAD // NS
HSTRIPE = 624
_SEGS = ((0, 128), (128, 128), (256, 128), (384, 128), (512, 112))
ORDER = (0, 5, 1, 6, 2, 7, 3, 8, 4, 9)
_IBLK = 32
_R0, _R1 = 160, 160

_mesh = plsc.VectorSubcoreMesh(core_axis_name="c", subcore_axis_name="s")


_BN1 = 400
_NB1 = NN // _BN1


def _mm_body(emb_ref, w_ref, b_ref, y_ref):
    x = emb_ref[...]
    for k in range(KK):
        y_ref[k] = (
            jnp.dot(x, w_ref[k], preferred_element_type=jnp.float32)
            + b_ref[k]
        )


def _mm_call(emb, W0, b0):
    return pl.pallas_call(
        _mm_body,
        grid=(_NB1,),
        in_specs=[
            pl.BlockSpec((_BN1, ED), lambda i: (i, 0)),
            pl.BlockSpec((KK, ED, ED), lambda i: (0, 0, 0)),
            pl.BlockSpec((KK, ED), lambda i: (0, 0)),
        ],
        out_specs=pl.BlockSpec((KK, _BN1, ED), lambda i: (0, i, 0)),
        out_shape=jax.ShapeDtypeStruct((KK, NN, ED), jnp.float32),
    )(emb, W0, b0)


@functools.partial(
    pl.kernel,
    out_type=jax.ShapeDtypeStruct((NC * CPAD,), jnp.float32),
    mesh=_mesh,
    scratch_types=[
        pltpu.VMEM((RPT, CHUNK), jnp.int32),
        pltpu.VMEM((CHUNK,), jnp.float32),
        pltpu.VMEM((ZR,), jnp.float32),
        pltpu.VMEM_SHARED((CPAD,), jnp.float32)
    ],
)
def _count_kernel(widx_hbm, out_hbm, idxbuf, ones_v, zbuf, c_sh):
    cid = lax.axis_index("c")
    sid = lax.axis_index("s")
    wid = sid * NC + cid
    zero16 = jnp.zeros((16,), jnp.float32)
    one16 = jnp.ones((16,), jnp.float32)

    def _zb(i, carry):
        zbuf[pl.ds(i * 16, 16)] = zero16
        return carry

    lax.fori_loop(0, ZR // 16, _zb, 0)
    for i in range(CHUNK // 16):
        ones_v[pl.ds(i * 16, 16)] = one16
    pltpu.sync_copy(zbuf, c_sh.at[pl.ds(sid * ZR, ZR)])
    plsc.subcore_barrier()

    pltpu.sync_copy(widx_hbm.at[pl.ds(wid * RPT, RPT)], idxbuf)

    def _body(j, carry):
        pltpu.sync_copy(ones_v, c_sh.at[idxbuf.at[j]], add=True)
        return carry

    lax.fori_loop(0, RPT, _body, 0)
    plsc.subcore_barrier()
    pltpu.sync_copy(c_sh.at[pl.ds(sid * ZR, ZR)], zbuf)
    pltpu.sync_copy(zbuf, out_hbm.at[pl.ds(cid * CPAD + sid * ZR, ZR)])


def _wt_body(cp_ref, w_ref):
    c = cp_ref[0] + cp_ref[1]
    a = 2.0 / jnp.maximum(c, 1.0)
    rows = [None] * KK
    p = jnp.ones((1, NN), jnp.float32)
    for j in reversed(range(KK)):
        kj = ORDER[j]
        p = p * a[kj:kj + 1, :]
        rows[kj] = p
    w_ref[...] = jnp.concatenate(rows, axis=0)


def _wt_call(cpr):
    return pl.pallas_call(
        _wt_body,
        grid=(1,),
        in_specs=[pl.BlockSpec((NC, KK, NN), lambda i: (0, 0, 0))],
        out_specs=pl.BlockSpec((KK, NN), lambda i: (0, 0)),
        out_shape=jax.ShapeDtypeStruct((KK, NN), jnp.float32),
    )(cpr)


_GDN = lax.GatherDimensionNumbers(
    offset_dims=(), collapsed_slice_dims=(0,), start_index_map=(0,))


def _bcast_lane(v16, i):
    return lax.gather(
        v16, jnp.full((16, 1), i, jnp.int32), _GDN, slice_sizes=(1,),
        mode=lax.GatherScatterMode.PROMISE_IN_BOUNDS)


@functools.partial(
    pl.kernel,
    out_type=jax.ShapeDtypeStruct((NC, NN, ED), jnp.float32),
    mesh=_mesh,
    scratch_types=[
        pltpu.VMEM((_IBLK, CHUNK), jnp.int32),
        pltpu.VMEM((_IBLK, CHUNK), jnp.int32),
        pltpu.VMEM((_IBLK, CHUNK), jnp.int32),
        pltpu.VMEM((2, CHUNK, ED), jnp.float32),
        pltpu.VMEM((2, CHUNK), jnp.float32),
        pltpu.VMEM_SHARED((NN, ED), jnp.float32),
        pltpu.SemaphoreType.DMA,
        pltpu.SemaphoreType.DMA,
        pltpu.SemaphoreType.DMA,
        pltpu.SemaphoreType.DMA,
        pltpu.SemaphoreType.DMA,
        pltpu.SemaphoreType.DMA,
    ],
)
def _scatter_kernel(yf_hbm, wflat_hbm, gidx_hbm, widx_hbm, didx_hbm, out_hbm,
                    gbuf, wibuf, dbuf, rows, wvals, h_sh,
                    sg0, sg1, sw0, sw1, ss0, ss1):
    cid = lax.axis_index("c")
    sid = lax.axis_index("s")
    wid = sid * NC + cid
    zero16 = jnp.zeros((16,), jnp.float32)
    sg = (sg0, sg1)
    sw = (sw0, sw1)
    ss = (ss0, ss1)

    def _issue_gather(jj, b):
        pltpu.async_copy(yf_hbm.at[gbuf.at[jj]], rows.at[b], sg[b])
        pltpu.async_copy(wflat_hbm.at[wibuf.at[jj]], wvals.at[b], sw[b])

    def _wait_gather(b):
        pltpu.make_async_copy(yf_hbm.at[gbuf.at[0]], rows.at[b], sg[b]).wait()
        pltpu.make_async_copy(wflat_hbm.at[wibuf.at[0]], wvals.at[b],
                              sw[b]).wait()

    def _issue_scatter(jj, b):
        pltpu.async_copy(rows.at[b], h_sh.at[dbuf.at[jj]], ss[b], add=True)

    def _wait_scatter(b):
        pltpu.make_async_copy(rows.at[b], h_sh.at[dbuf.at[0]], ss[b]).wait()

    def _scale(b):
        def _grp(g, c2):
            wv = wvals[b, pl.ds(g * 16, 16)]
            for i in range(16):
                wb = _bcast_lane(wv, i)
                e = g * 16 + i
                for cb in range(ED // 16):
                    sl = pl.ds(cb * 16, 16)
                    rows[b, e, sl] = rows[b, e, sl] * wb
            return c2

        lax.fori_loop(0, CHUNK // 16, _grp, 0)

    def _zrow(r, carry):
        for cb in range(ED // 16):
            rows[0, r, pl.ds(cb * 16, 16)] = zero16
        return carry

    lax.fori_loop(0, CHUNK, _zrow, 0)
    hbase = sid * HSTRIPE
    for off, sz in _SEGS:
        pltpu.sync_copy(rows.at[0, pl.ds(0, sz)],
                        h_sh.at[pl.ds(hbase + off, sz)])

    @pl.when(sid == NS - 1)
    def _zero_tail():
        pltpu.sync_copy(rows.at[0, pl.ds(0, 16)], h_sh.at[pl.ds(NN - 16, 16)])

    plsc.subcore_barrier()

    row0 = jnp.where(cid == 0, sid * _R0, NS * _R0 + sid * _R1)
    nblk = jnp.where(cid == 0, _R0 // _IBLK, _R1 // _IBLK)

    def _iblk(bi, carry):
        rb = row0 + bi * _IBLK

        @pl.when(bi >= 1)
        def _wait_prev_tail():
            _wait_scatter(1)

        pltpu.sync_copy(gidx_hbm.at[pl.ds(rb, _IBLK)], gbuf)
        pltpu.sync_copy(widx_hbm.at[pl.ds(rb, _IBLK)], wibuf)
        pltpu.sync_copy(didx_hbm.at[pl.ds(rb, _IBLK)], dbuf)
        _issue_gather(0, 0)

        def _pair(p, c1):
            @pl.when(p >= 1)
            def _w0():
                _wait_scatter(1)

            _issue_gather(2 * p + 1, 1)
            _wait_gather(0)
            _scale(0)
            _issue_scatter(2 * p, 0)
            _wait_scatter(0)

            @pl.when(p <= _IBLK // 2 - 2)
            def _pf1():
                _issue_gather(2 * p + 2, 0)

            _wait_gather(1)
            _scale(1)
            _issue_scatter(2 * p + 1, 1)
            return c1

        lax.fori_loop(0, _IBLK // 2, _pair, 0)
        return carry

    lax.fori_loop(0, nblk, _iblk, 0)
    _wait_scatter(1)
    plsc.subcore_barrier()
    for off, sz in _SEGS:
        sl = pl.ds(hbase + off, sz)
        pltpu.sync_copy(h_sh.at[sl], rows.at[0, pl.ds(0, sz)])
        pltpu.sync_copy(rows.at[0, pl.ds(0, sz)], out_hbm.at[cid, sl])

    @pl.when(sid == NS - 1)
    def _out_tail():
        sl = pl.ds(NN - 16, 16)
        pltpu.sync_copy(h_sh.at[sl], rows.at[0, pl.ds(0, 16)])
        pltpu.sync_copy(rows.at[0, pl.ds(0, 16)], out_hbm.at[cid, sl])


_BN5 = 400
_NB5 = NN // _BN5


def _comb_body(emb_ref, wemb_ref, hp_ref, out_ref):
    out_ref[...] = emb_ref[...] * wemb_ref[...] + hp_ref[0] + hp_ref[1]


def _comb_call(emb, wemb, hp):
    return pl.pallas_call(
        _comb_body,
        grid=(_NB5,),
        in_specs=[
            pl.BlockSpec((_BN5, ED), lambda i: (i, 0)),
            pl.BlockSpec((_BN5, 1), lambda i: (i, 0)),
            pl.BlockSpec((NC, _BN5, ED), lambda i: (0, i, 0)),
        ],
        out_specs=pl.BlockSpec((_BN5, ED), lambda i: (i, 0)),
        out_shape=jax.ShapeDtypeStruct((NN, ED), jnp.float32),
    )(emb, wemb, hp)


def kernel(edge_index, edge_type, embeddings, W0, b0):
    ne = edge_index.shape[1]
    t = edge_type.astype(jnp.int32)
    ei0 = edge_index[0].astype(jnp.int32)
    ei1 = edge_index[1].astype(jnp.int32)

    gidx = jnp.concatenate([t * NN + ei1, (t + NR) * NN + ei0])
    widx = jnp.concatenate([t * NN + ei0, (t + NR) * NN + ei1])
    didx = jnp.concatenate([ei0, ei1])
    pad = NOP - 2 * ne
    spread = lax.iota(jnp.int32, pad)
    gidx = jnp.concatenate([gidx, spread % (KK * NN)])
    widx = jnp.concatenate([widx, jnp.full((pad,), KK * NN, jnp.int32)])
    didx = jnp.concatenate([didx, spread % NN])
    gidx2 = gidx.reshape(NOPROWS, CHUNK)
    widx2 = widx.reshape(NOPROWS, CHUNK)
    didx2 = didx.reshape(NOPROWS, CHUNK)

    yf = _mm_call(embeddings, W0, b0).reshape(KK * NN, ED)
    cp = _count_kernel(widx2).reshape(NC, CPAD)
    cpr = cp[:, :KK * NN].reshape(NC, KK, NN)
    w = _wt_call(cpr)
    wemb = w[0].reshape(NN, 1)
    wflat = jnp.concatenate(
        [w.reshape(-1), jnp.zeros((CPAD - KK * NN,), jnp.float32)])
    hp = _scatter_kernel(yf, wflat, gidx2, widx2, didx2)
    return _comb_call(embeddings, wemb, hp)


# speedup vs baseline: 28.0487x; 1.0106x over previous
import functools

import jax
import jax.numpy as jnp
from jax import lax
from jax.experimental import pallas as pl
from jax.experimental.pallas import tpu as pltpu
from jax.experimental.pallas import tpu_sc as plsc

NN = 10000
NR = 5
KK = 2 * NR
ED = 128
NC, NS, LL = 2, 16, 16
NW = NC * NS
CHUNK = 128
RPT = 160
NOP = NW * RPT * CHUNK
NOPROWS = NOP // CHUNK
CPAD = KK * NN + 96
ZR = CPAD // NS
HSTRIPE = 624
_SEGS = ((0, 128), (128, 128), (256, 128), (384, 128), (512, 112))
ORDER = (0, 5, 1, 6, 2, 7, 3, 8, 4, 9)
_IBLK = 40
_R0, _R1 = 160, 160

_mesh = plsc.VectorSubcoreMesh(core_axis_name="c", subcore_axis_name="s")


_BN1 = 400
_NB1 = NN // _BN1


def _mm_body(emb_ref, w_ref, b_ref, y_ref):
    x = emb_ref[...]
    for k in range(KK):
        y_ref[k] = (
            jnp.dot(x, w_ref[k], preferred_element_type=jnp.float32)
            + b_ref[k]
        )


def _mm_call(emb, W0, b0):
    return pl.pallas_call(
        _mm_body,
        grid=(_NB1,),
        in_specs=[
            pl.BlockSpec((_BN1, ED), lambda i: (i, 0)),
            pl.BlockSpec((KK, ED, ED), lambda i: (0, 0, 0)),
            pl.BlockSpec((KK, ED), lambda i: (0, 0)),
        ],
        out_specs=pl.BlockSpec((KK, _BN1, ED), lambda i: (0, i, 0)),
        out_shape=jax.ShapeDtypeStruct((KK, NN, ED), jnp.float32),
    )(emb, W0, b0)


@functools.partial(
    pl.kernel,
    out_type=jax.ShapeDtypeStruct((NC * CPAD,), jnp.float32),
    mesh=_mesh,
    scratch_types=[
        pltpu.VMEM((RPT, CHUNK), jnp.int32),
        pltpu.VMEM((CHUNK,), jnp.float32),
        pltpu.VMEM((ZR,), jnp.float32),
        pltpu.VMEM_SHARED((CPAD,), jnp.float32)
    ],
)
def _count_kernel(widx_hbm, out_hbm, idxbuf, ones_v, zbuf, c_sh):
    cid = lax.axis_index("c")
    sid = lax.axis_index("s")
    wid = sid * NC + cid
    zero16 = jnp.zeros((16,), jnp.float32)
    one16 = jnp.ones((16,), jnp.float32)

    def _zb(i, carry):
        zbuf[pl.ds(i * 16, 16)] = zero16
        return carry

    lax.fori_loop(0, ZR // 16, _zb, 0)
    for i in range(CHUNK // 16):
        ones_v[pl.ds(i * 16, 16)] = one16
    pltpu.sync_copy(zbuf, c_sh.at[pl.ds(sid * ZR, ZR)])
    plsc.subcore_barrier()

    pltpu.sync_copy(widx_hbm.at[pl.ds(wid * RPT, RPT)], idxbuf)

    def _body(j, carry):
        pltpu.sync_copy(ones_v, c_sh.at[idxbuf.at[j]], add=True)
        return carry

    lax.fori_loop(0, RPT, _body, 0)
    plsc.subcore_barrier()
    pltpu.sync_copy(c_sh.at[pl.ds(sid * ZR, ZR)], zbuf)
    pltpu.sync_copy(zbuf, out_hbm.at[pl.ds(cid * CPAD + sid * ZR, ZR)])


def _wt_body(cp_ref, w_ref):
    c = cp_ref[0] + cp_ref[1]
    a = 2.0 / jnp.maximum(c, 1.0)
    rows = [None] * KK
    p = jnp.ones((1, NN), jnp.float32)
    for j in reversed(range(KK)):
        kj = ORDER[j]
        p = p * a[kj:kj + 1, :]
        rows[kj] = p
    w_ref[...] = jnp.concatenate(rows, axis=0)


def _wt_call(cpr):
    return pl.pallas_call(
        _wt_body,
        grid=(1,),
        in_specs=[pl.BlockSpec((NC, KK, NN), lambda i: (0, 0, 0))],
        out_specs=pl.BlockSpec((KK, NN), lambda i: (0, 0)),
        out_shape=jax.ShapeDtypeStruct((KK, NN), jnp.float32),
    )(cpr)


_GDN = lax.GatherDimensionNumbers(
    offset_dims=(), collapsed_slice_dims=(0,), start_index_map=(0,))


def _bcast_lane(v16, i):
    return lax.gather(
        v16, jnp.full((16, 1), i, jnp.int32), _GDN, slice_sizes=(1,),
        mode=lax.GatherScatterMode.PROMISE_IN_BOUNDS)


@functools.partial(
    pl.kernel,
    out_type=jax.ShapeDtypeStruct((NC, NN, ED), jnp.float32),
    mesh=_mesh,
    scratch_types=[
        pltpu.VMEM((_IBLK, CHUNK), jnp.int32),
        pltpu.VMEM((_IBLK, CHUNK), jnp.int32),
        pltpu.VMEM((_IBLK, CHUNK), jnp.int32),
        pltpu.VMEM((2, CHUNK, ED), jnp.float32),
        pltpu.VMEM((2, CHUNK), jnp.float32),
        pltpu.VMEM_SHARED((NN, ED), jnp.float32),
        pltpu.SemaphoreType.DMA,
        pltpu.SemaphoreType.DMA,
        pltpu.SemaphoreType.DMA,
        pltpu.SemaphoreType.DMA,
        pltpu.SemaphoreType.DMA,
        pltpu.SemaphoreType.DMA,
    ],
)
def _scatter_kernel(yf_hbm, wflat_hbm, gidx_hbm, widx_hbm, didx_hbm, out_hbm,
                    gbuf, wibuf, dbuf, rows, wvals, h_sh,
                    sg0, sg1, sw0, sw1, ss0, ss1):
    cid = lax.axis_index("c")
    sid = lax.axis_index("s")
    wid = sid * NC + cid
    zero16 = jnp.zeros((16,), jnp.float32)
    sg = (sg0, sg1)
    sw = (sw0, sw1)
    ss = (ss0, ss1)

    def _issue_gather(jj, b):
        pltpu.async_copy(yf_hbm.at[gbuf.at[jj]], rows.at[b], sg[b])
        pltpu.async_copy(wflat_hbm.at[wibuf.at[jj]], wvals.at[b], sw[b])

    def _wait_gather(b):
        pltpu.make_async_copy(yf_hbm.at[gbuf.at[0]], rows.at[b], sg[b]).wait()
        pltpu.make_async_copy(wflat_hbm.at[wibuf.at[0]], wvals.at[b],
                              sw[b]).wait()

    def _issue_scatter(jj, b):
        pltpu.async_copy(rows.at[b], h_sh.at[dbuf.at[jj]], ss[b], add=True)

    def _wait_scatter(b):
        pltpu.make_async_copy(rows.at[b], h_sh.at[dbuf.at[0]], ss[b]).wait()

    def _scale(b):
        def _grp(g, c2):
            wv = wvals[b, pl.ds(g * 16, 16)]
            for i in range(16):
                wb = _bcast_lane(wv, i)
                e = g * 16 + i
                for cb in range(ED // 16):
                    sl = pl.ds(cb * 16, 16)
                    rows[b, e, sl] = rows[b, e, sl] * wb
            return c2

        lax.fori_loop(0, CHUNK // 16, _grp, 0)

    def _zrow(r, carry):
        for cb in range(ED // 16):
            rows[0, r, pl.ds(cb * 16, 16)] = zero16
        return carry

    lax.fori_loop(0, CHUNK, _zrow, 0)
    hbase = sid * HSTRIPE
    for off, sz in _SEGS:
        pltpu.sync_copy(rows.at[0, pl.ds(0, sz)],
                        h_sh.at[pl.ds(hbase + off, sz)])

    @pl.when(sid == NS - 1)
    def _zero_tail():
        pltpu.sync_copy(rows.at[0, pl.ds(0, 16)], h_sh.at[pl.ds(NN - 16, 16)])

    plsc.subcore_barrier()

    row0 = jnp.where(cid == 0, sid * _R0, NS * _R0 + sid * _R1)
    nblk = jnp.where(cid == 0, _R0 // _IBLK, _R1 // _IBLK)

    def _iblk(bi, carry):
        rb = row0 + bi * _IBLK

        @pl.when(bi >= 1)
        def _wait_prev_tail():
            _wait_scatter(1)

        pltpu.sync_copy(gidx_hbm.at[pl.ds(rb, _IBLK)], gbuf)
        pltpu.sync_copy(widx_hbm.at[pl.ds(rb, _IBLK)], wibuf)
        pltpu.sync_copy(didx_hbm.at[pl.ds(rb, _IBLK)], dbuf)
        _issue_gather(0, 0)

        def _pair(p, c1):
            @pl.when(p >= 1)
            def _w0():
                _wait_scatter(1)

            _issue_gather(2 * p + 1, 1)
            _wait_gather(0)
            _scale(0)
            _issue_scatter(2 * p, 0)
            _wait_scatter(0)

            @pl.when(p <= _IBLK // 2 - 2)
            def _pf1():
                _issue_gather(2 * p + 2, 0)

            _wait_gather(1)
            _scale(1)
            _issue_scatter(2 * p + 1, 1)
            return c1

        lax.fori_loop(0, _IBLK // 2, _pair, 0)
        return carry

    lax.fori_loop(0, nblk, _iblk, 0)
    _wait_scatter(1)
    plsc.subcore_barrier()
    for off, sz in _SEGS:
        sl = pl.ds(hbase + off, sz)
        pltpu.sync_copy(h_sh.at[sl], rows.at[0, pl.ds(0, sz)])
        pltpu.sync_copy(rows.at[0, pl.ds(0, sz)], out_hbm.at[cid, sl])

    @pl.when(sid == NS - 1)
    def _out_tail():
        sl = pl.ds(NN - 16, 16)
        pltpu.sync_copy(h_sh.at[sl], rows.at[0, pl.ds(0, 16)])
        pltpu.sync_copy(rows.at[0, pl.ds(0, 16)], out_hbm.at[cid, sl])


_BN5 = 400
_NB5 = NN // _BN5


def _comb_body(emb_ref, wemb_ref, hp_ref, out_ref):
    out_ref[...] = emb_ref[...] * wemb_ref[...] + hp_ref[0] + hp_ref[1]


def _comb_call(emb, wemb, hp):
    return pl.pallas_call(
        _comb_body,
        grid=(_NB5,),
        in_specs=[
            pl.BlockSpec((_BN5, ED), lambda i: (i, 0)),
            pl.BlockSpec((_BN5, 1), lambda i: (i, 0)),
            pl.BlockSpec((NC, _BN5, ED), lambda i: (0, i, 0)),
        ],
        out_specs=pl.BlockSpec((_BN5, ED), lambda i: (i, 0)),
        out_shape=jax.ShapeDtypeStruct((NN, ED), jnp.float32),
    )(emb, wemb, hp)


def kernel(edge_index, edge_type, embeddings, W0, b0):
    ne = edge_index.shape[1]
    t = edge_type.astype(jnp.int32)
    ei0 = edge_index[0].astype(jnp.int32)
    ei1 = edge_index[1].astype(jnp.int32)

    gidx = jnp.concatenate([t * NN + ei1, (t + NR) * NN + ei0])
    widx = jnp.concatenate([t * NN + ei0, (t + NR) * NN + ei1])
    didx = jnp.concatenate([ei0, ei1])
    pad = NOP - 2 * ne
    spread = lax.iota(jnp.int32, pad)
    gidx = jnp.concatenate([gidx, spread % (KK * NN)])
    widx = jnp.concatenate([widx, jnp.full((pad,), KK * NN, jnp.int32)])
    didx = jnp.concatenate([didx, spread % NN])
    gidx2 = gidx.reshape(NOPROWS, CHUNK)
    widx2 = widx.reshape(NOPROWS, CHUNK)
    didx2 = didx.reshape(NOPROWS, CHUNK)

    yf = _mm_call(embeddings, W0, b0).reshape(KK * NN, ED)
    cp = _count_kernel(widx2).reshape(NC, CPAD)
    cpr = cp[:, :KK * NN].reshape(NC, KK, NN)
    w = _wt_call(cpr)
    wemb = w[0].reshape(NN, 1)
    wflat = jnp.concatenate(
        [w.reshape(-1), jnp.zeros((CPAD - KK * NN,), jnp.float32)])
    hp = _scatter_kernel(yf, wflat, gidx2, widx2, didx2)
    return _comb_call(embeddings, wemb, hp)
